# Initial kernel scaffold; baseline (speedup 1.0000x reference)
import jax
import jax.numpy as jnp
from jax.experimental import pallas as pl


def kernel(h_node, h_edge, edge_index, node_extra, edge_extra, params):
    raise NotImplementedError("write your pallas kernel here")


# speedup vs baseline: 2.6995x; 2.6995x over previous
import functools

import jax
import jax.numpy as jnp
from jax import lax
from jax.experimental import pallas as pl
from jax.experimental.pallas import tpu as pltpu
from jax.experimental.pallas import tpu_sc as plsc

N_NODES = 10000
N_EDGES = 160000
NODE_DIM = 128
EDGE_DIM = 16
INTER = 32

NC, NS = 2, 16
NW = NC * NS
BATCH = 125
EPW = N_EDGES // NW
NB = EPW // BATCH
N_PAD = 10240
ROWS_PT = N_PAD // NS
ZCH = 40

_MESH = plsc.VectorSubcoreMesh(core_axis_name="c", subcore_axis_name="s")


def _f32(*shape):
    return jax.ShapeDtypeStruct(shape, jnp.float32)


def _fill(ref, rows, width, val):
    def body(r, _):
        for cidx in range(width // 16):
            ref[r, pl.ds(cidx * 16, 16)] = jnp.full((16,), val, jnp.float32)
        return 0
    lax.fori_loop(0, rows, body, 0)


def _worker_id():
    return lax.axis_index("s") * NC + lax.axis_index("c")


def _make_sc_deg():
    def body(row_i, out, iR, buf, zb, acc, sem):
        c = lax.axis_index("c")
        s = lax.axis_index("s")
        w = s * NC + c
        _fill(zb, ZCH, 16, 0.0)
        for z in range(ROWS_PT // ZCH):
            pltpu.sync_copy(zb, acc.at[pl.ds(s * ROWS_PT + z * ZCH, ZCH)])
        plsc.subcore_barrier()
        _fill(buf, BATCH, 16, 1.0)
        pltpu.sync_copy(row_i.at[w], iR)

        def step(j, _):
            pltpu.sync_copy(buf, acc.at[iR.at[j]], add=True)
            return 0
        lax.fori_loop(0, NB, step, 0)
        plsc.subcore_barrier()
        pltpu.sync_copy(acc.at[pl.ds(s * ROWS_PT, ROWS_PT)],
                        out.at[pl.ds(c * N_PAD + s * ROWS_PT, ROWS_PT)])

    return pl.kernel(
        body,
        compiler_params=pltpu.CompilerParams(use_tc_tiling_on_sc=False),
        out_type=_f32(2 * N_PAD, 16),
        mesh=_MESH,
        scratch_types=[
            pltpu.VMEM((NB, BATCH), jnp.int32),
            pltpu.VMEM((BATCH, 16), jnp.float32),
            pltpu.VMEM((ZCH, 16), jnp.float32),
            pltpu.VMEM_SHARED((N_PAD, 16), jnp.float32),
            pltpu.SemaphoreType.DMA,
        ],
    )


def _make_sc_node_agg():
    def body(he3, hn, row_i, col_i, out, rV, cV, bufA, bufB, zb, acc,
             semA, semB):
        c = lax.axis_index("c")
        s = lax.axis_index("s")
        w = s * NC + c
        _fill(zb, ZCH, NODE_DIM, 0.0)
        for z in range(ROWS_PT // ZCH):
            pltpu.sync_copy(zb, acc.at[pl.ds(s * ROWS_PT + z * ZCH, ZCH)])
        plsc.subcore_barrier()
        pltpu.sync_copy(row_i.at[w], rV)
        pltpu.sync_copy(col_i.at[w], cV)

        def step(j, _):
            dA = pltpu.async_copy(he3.at[w * NB + j], bufA, semA)
            dB = pltpu.async_copy(hn.at[cV.at[j]], bufB, semB)
            dA.wait()
            dB.wait()
            pltpu.sync_copy(bufA, acc.at[rV.at[j]], add=True)
            pltpu.sync_copy(bufB, acc.at[rV.at[j]], add=True)
            return 0
        lax.fori_loop(0, NB, step, 0)
        plsc.subcore_barrier()
        pltpu.sync_copy(acc.at[pl.ds(s * ROWS_PT, ROWS_PT)],
                        out.at[pl.ds(c * N_PAD + s * ROWS_PT, ROWS_PT)])

    return pl.kernel(
        body,
        compiler_params=pltpu.CompilerParams(use_tc_tiling_on_sc=False),
        out_type=_f32(2 * N_PAD, NODE_DIM),
        mesh=_MESH,
        scratch_types=[
            pltpu.VMEM((NB, BATCH), jnp.int32),
            pltpu.VMEM((NB, BATCH), jnp.int32),
            pltpu.VMEM((BATCH, NODE_DIM), jnp.float32),
            pltpu.VMEM((BATCH, NODE_DIM), jnp.float32),
            pltpu.VMEM((ZCH, NODE_DIM), jnp.float32),
            pltpu.VMEM_SHARED((N_PAD, NODE_DIM), jnp.float32),
            pltpu.SemaphoreType.DMA,
            pltpu.SemaphoreType.DMA,
        ],
    )


def _make_sc_gather2(wa, wb):
    def body(tabA, tabB, idxA, idxB, gA, gB, iA, iB, bufA, bufB, semA, semB):
        c = lax.axis_index("c")
        s = lax.axis_index("s")
        w = s * NC + c
        pltpu.sync_copy(idxA.at[w], iA)
        pltpu.sync_copy(idxB.at[w], iB)

        def step(j, _):
            dA = pltpu.async_copy(tabA.at[iA.at[j]], bufA, semA)
            dB = pltpu.async_copy(tabB.at[iB.at[j]], bufB, semB)
            dA.wait()
            pltpu.sync_copy(bufA, gA.at[w * NB + j])
            dB.wait()
            pltpu.sync_copy(bufB, gB.at[w * NB + j])
            return 0
        lax.fori_loop(0, NB, step, 0)

    return pl.kernel(
        body,
        compiler_params=pltpu.CompilerParams(use_tc_tiling_on_sc=False),
        out_type=(_f32(NW * NB, BATCH, wa), _f32(NW * NB, BATCH, wb)),
        mesh=_MESH,
        scratch_types=[
            pltpu.VMEM((NB, BATCH), jnp.int32),
            pltpu.VMEM((NB, BATCH), jnp.int32),
            pltpu.VMEM((BATCH, wa), jnp.float32),
            pltpu.VMEM((BATCH, wb), jnp.float32),
            pltpu.SemaphoreType.DMA,
            pltpu.SemaphoreType.DMA,
        ],
    )


def _make_sc_scatter2():
    def body(vA3, vB3, idxA, idxB, oA, oB, iA, iB, bufA, bufB, zb,
             accA, accB, semA, semB):
        c = lax.axis_index("c")
        s = lax.axis_index("s")
        w = s * NC + c
        _fill(zb, ZCH, 16, 0.0)
        for z in range(ROWS_PT // ZCH):
            pltpu.sync_copy(zb, accA.at[pl.ds(s * ROWS_PT + z * ZCH, ZCH)])
            pltpu.sync_copy(zb, accB.at[pl.ds(s * ROWS_PT + z * ZCH, ZCH)])
        plsc.subcore_barrier()
        pltpu.sync_copy(idxA.at[w], iA)
        pltpu.sync_copy(idxB.at[w], iB)

        def step(j, _):
            dA = pltpu.async_copy(vA3.at[w * NB + j], bufA, semA)
            dB = pltpu.async_copy(vB3.at[w * NB + j], bufB, semB)
            dA.wait()
            pltpu.sync_copy(bufA, accA.at[iA.at[j]], add=True)
            dB.wait()
            pltpu.sync_copy(bufB, accB.at[iB.at[j]], add=True)
            return 0
        lax.fori_loop(0, NB, step, 0)
        plsc.subcore_barrier()
        pltpu.sync_copy(accA.at[pl.ds(s * ROWS_PT, ROWS_PT)],
                        oA.at[pl.ds(c * N_PAD + s * ROWS_PT, ROWS_PT)])
        pltpu.sync_copy(accB.at[pl.ds(s * ROWS_PT, ROWS_PT)],
                        oB.at[pl.ds(c * N_PAD + s * ROWS_PT, ROWS_PT)])

    return pl.kernel(
        body,
        compiler_params=pltpu.CompilerParams(use_tc_tiling_on_sc=False),
        out_type=(_f32(2 * N_PAD, 16), _f32(2 * N_PAD, 16)),
        mesh=_MESH,
        scratch_types=[
            pltpu.VMEM((NB, BATCH), jnp.int32),
            pltpu.VMEM((NB, BATCH), jnp.int32),
            pltpu.VMEM((BATCH, 16), jnp.float32),
            pltpu.VMEM((BATCH, 16), jnp.float32),
            pltpu.VMEM((ZCH, 16), jnp.float32),
            pltpu.VMEM_SHARED((N_PAD, 16), jnp.float32),
            pltpu.VMEM_SHARED((N_PAD, 16), jnp.float32),
            pltpu.SemaphoreType.DMA,
            pltpu.SemaphoreType.DMA,
        ],
    )


_sc_deg = _make_sc_deg()
_sc_node_agg = _make_sc_node_agg()
_sc_gather32 = _make_sc_gather2(INTER, INTER)
_sc_gather16 = _make_sc_gather2(16, 16)
_sc_scatter2 = _make_sc_scatter2()


def _ln(x, g, b):
    m = jnp.mean(x, axis=-1, keepdims=True)
    v = jnp.mean((x - m) ** 2, axis=-1, keepdims=True)
    return (x - m) * lax.rsqrt(v + 1e-5) * g + b


def _dot(x, w):
    return jnp.dot(x, w, preferred_element_type=jnp.float32)


BN = 2000
BE = 2000


def _wspec(r, c):
    return pl.BlockSpec((r, c), lambda i: (0, 0))


def _rspec(rows, cols):
    return pl.BlockSpec((rows, cols), lambda i: (i, 0))


def _r3spec(rows, cols, half):
    return pl.BlockSpec((1, rows, cols), lambda i, h=half: (h, i, 0))


def _tc_mlp_kernel(x, w0, b0, g, bb, w1, b1, o):
    h = _dot(x[...], w0[...]) + b0[...]
    h = jnp.maximum(_ln(h, g[...], bb[...]), 0.0)
    o[...] = _dot(h, w1[...]) + b1[...]


def _tc_node_mlp(x, w0, b0, g, bb, w1, b1):
    return pl.pallas_call(
        _tc_mlp_kernel,
        grid=(N_NODES // BN,),
        in_specs=[_rspec(BN, NODE_DIM), _wspec(NODE_DIM, NODE_DIM),
                  _wspec(1, NODE_DIM), _wspec(1, NODE_DIM),
                  _wspec(1, NODE_DIM), _wspec(NODE_DIM, NODE_DIM),
                  _wspec(1, NODE_DIM)],
        out_specs=_rspec(BN, NODE_DIM),
        out_shape=_f32(N_NODES, NODE_DIM),
    )(x, w0, b0, g, bb, w1, b1)


def _tc_edge_mlp(x, w0, b0, g, bb, w1, b1):
    return pl.pallas_call(
        _tc_mlp_kernel,
        grid=(N_EDGES // BE,),
        in_specs=[_rspec(BE, EDGE_DIM), _wspec(EDGE_DIM, NODE_DIM),
                  _wspec(1, NODE_DIM), _wspec(1, NODE_DIM),
                  _wspec(1, NODE_DIM), _wspec(NODE_DIM, NODE_DIM),
                  _wspec(1, NODE_DIM)],
        out_specs=_rspec(BE, NODE_DIM),
        out_shape=_f32(N_EDGES, NODE_DIM),
    )(x, w0, b0, g, bb, w1, b1)


def _tc_node_finish_kernel(x, hn, seg_lo, seg_hi, deg_lo, deg_hi,
                           wm, bm, wc, bc, o0, ob0, og, obb, o1, ob1, fg, fb,
                           wtl, wtr, wcbl, wcbr,
                           xn, tl, tr, cbl, cbr):
    seg = seg_lo[0] + seg_hi[0]
    deg = deg_lo[0][:, :1] + deg_hi[0][:, :1]
    xb = x[...]
    u = seg + deg * hn[...]
    aggr = _dot(u, wm[...]) + deg * bm[...]
    t = _dot(xb, wc[...]) + bc[...] + aggr
    h = _dot(t, o0[...]) + ob0[...]
    h = jnp.maximum(_ln(h, og[...], obb[...]), 0.0)
    t2 = _dot(h, o1[...]) + ob1[...]
    y = _ln(t2 + xb, fg[...], fb[...])
    xn[...] = y
    tl[...] = _dot(y, wtl[...])
    tr[...] = _dot(y, wtr[...])
    cbl[...] = _dot(y, wcbl[...])
    cbr[...] = _dot(y, wcbr[...])


def _tc_node_finish(x, hn, seg2, deg2, wm, bm, wc, bc, o0, ob0, og, obb,
                    o1, ob1, fg, fb, wtl, wtr, wcbl, wcbr):
    return pl.pallas_call(
        _tc_node_finish_kernel,
        grid=(N_NODES // BN,),
        in_specs=[_rspec(BN, NODE_DIM), _rspec(BN, NODE_DIM),
                  _r3spec(BN, NODE_DIM, 0), _r3spec(BN, NODE_DIM, 1),
                  _r3spec(BN, 16, 0), _r3spec(BN, 16, 1),
                  _wspec(NODE_DIM, NODE_DIM), _wspec(1, NODE_DIM),
                  _wspec(NODE_DIM, NODE_DIM), _wspec(1, NODE_DIM),
                  _wspec(NODE_DIM, NODE_DIM), _wspec(1, NODE_DIM),
                  _wspec(1, NODE_DIM), _wspec(1, NODE_DIM),
                  _wspec(NODE_DIM, NODE_DIM), _wspec(1, NODE_DIM),
                  _wspec(1, NODE_DIM), _wspec(1, NODE_DIM),
                  _wspec(NODE_DIM, INTER), _wspec(NODE_DIM, INTER),
                  _wspec(NODE_DIM, 16), _wspec(NODE_DIM, 16)],
        out_specs=[_rspec(BN, NODE_DIM), _rspec(BN, INTER), _rspec(BN, INTER),
                   _rspec(BN, 16), _rspec(BN, 16)],
        out_shape=(_f32(N_NODES, NODE_DIM), _f32(N_NODES, INTER),
                   _f32(N_NODES, INTER), _f32(N_NODES, 16),
                   _f32(N_NODES, 16)),
    )(x, hn, seg2, seg2, deg2, deg2, wm, bm, wc, bc, o0, ob0, og, obb,
      o1, ob1, fg, fb, wtl, wtr, wcbl, wcbr)


def _tc_edge_inter_kernel(hb, gl, gr, bl, l0, lb0, lg, lbb, l1, lb1,
                          br, r0, rb0, rg, rbb, r1, rb1, mbl, mbr):
    hbb = hb[...]
    il = _dot(hbb, bl[...]) + gl[...]
    h = _dot(il, l0[...]) + lb0[...]
    h = jnp.maximum(_ln(h, lg[...], lbb[...]), 0.0)
    mbl[...] = _dot(h, l1[...]) + lb1[...]
    ir = _dot(hbb, br[...]) + gr[...]
    h = _dot(ir, r0[...]) + rb0[...]
    h = jnp.maximum(_ln(h, rg[...], rbb[...]), 0.0)
    mbr[...] = _dot(h, r1[...]) + rb1[...]


def _tc_edge_inter(hb, gl, gr, *ws):
    return pl.pallas_call(
        _tc_edge_inter_kernel,
        grid=(N_EDGES // BE,),
        in_specs=[_rspec(BE, EDGE_DIM), _rspec(BE, INTER), _rspec(BE, INTER),
                  _wspec(EDGE_DIM, INTER), _wspec(INTER, INTER),
                  _wspec(1, INTER), _wspec(1, INTER), _wspec(1, INTER),
                  _wspec(INTER, EDGE_DIM), _wspec(1, EDGE_DIM),
                  _wspec(EDGE_DIM, INTER), _wspec(INTER, INTER),
                  _wspec(1, INTER), _wspec(1, INTER), _wspec(1, INTER),
                  _wspec(INTER, EDGE_DIM), _wspec(1, EDGE_DIM)],
        out_specs=[_rspec(BE, EDGE_DIM), _rspec(BE, EDGE_DIM)],
        out_shape=(_f32(N_EDGES, EDGE_DIM), _f32(N_EDGES, EDGE_DIM)),
    )(hb, gl, gr, *ws)


def _tc_node_mix_kernel(sl_lo, sl_hi, sr_lo, sr_hi, cbl, cbr, ml, mr,
                        cL, cR):
    cL[...] = _dot(sl_lo[0] + sl_hi[0], ml[...]) + cbl[...]
    cR[...] = _dot(sr_lo[0] + sr_hi[0], mr[...]) + cbr[...]


def _tc_node_mix(sl2, sr2, cbl, cbr, ml, mr):
    return pl.pallas_call(
        _tc_node_mix_kernel,
        grid=(N_NODES // BN,),
        in_specs=[_r3spec(BN, 16, 0), _r3spec(BN, 16, 1),
                  _r3spec(BN, 16, 0), _r3spec(BN, 16, 1),
                  _rspec(BN, 16), _rspec(BN, 16),
                  _wspec(16, 16), _wspec(16, 16)],
        out_specs=[_rspec(BN, 16), _rspec(BN, 16)],
        out_shape=(_f32(N_NODES, 16), _f32(N_NODES, 16)),
    )(sl2, sl2, sr2, sr2, cbl, cbr, ml, mr)


def _tc_edge_finish_kernel(gL, gR, hb, ws, ball, o0, ob0, og, obb, o1, ob1,
                           fg, fb, out):
    hbb = hb[...]
    upd = gL[...] + gR[...] + _dot(hbb, ws[...]) + ball[...]
    h = _dot(upd, o0[...]) + ob0[...]
    h = jnp.maximum(_ln(h, og[...], obb[...]), 0.0)
    t = _dot(h, o1[...]) + ob1[...]
    out[...] = _ln(t + hbb, fg[...], fb[...])


def _tc_edge_finish(gL, gR, hb, *ws):
    return pl.pallas_call(
        _tc_edge_finish_kernel,
        grid=(N_EDGES // BE,),
        in_specs=[_rspec(BE, EDGE_DIM), _rspec(BE, EDGE_DIM),
                  _rspec(BE, EDGE_DIM),
                  _wspec(EDGE_DIM, EDGE_DIM), _wspec(1, EDGE_DIM),
                  _wspec(EDGE_DIM, EDGE_DIM), _wspec(1, EDGE_DIM),
                  _wspec(1, EDGE_DIM), _wspec(1, EDGE_DIM),
                  _wspec(EDGE_DIM, EDGE_DIM), _wspec(1, EDGE_DIM),
                  _wspec(1, EDGE_DIM), _wspec(1, EDGE_DIM)],
        out_specs=_rspec(BE, EDGE_DIM),
        out_shape=_f32(N_EDGES, EDGE_DIM),
    )(gL, gR, hb, *ws)


def _r1(v):
    return v.reshape(1, -1)


def _mlp_ws(p):
    return (p["l0"]["w"], _r1(p["l0"]["b"]), _r1(p["ln"]["g"]),
            _r1(p["ln"]["b"]), p["l1"]["w"], _r1(p["l1"]["b"]))


def kernel(h_node, h_edge, edge_index, node_extra, edge_extra, params):
    row3 = edge_index[0].reshape(NW, NB, BATCH)
    col3 = edge_index[1].reshape(NW, NB, BATCH)

    deg2 = _sc_deg(row3).reshape(2, N_PAD, 16)

    x, hb = h_node, h_edge
    for blk in params["blocks"]:
        npar, epar = blk["node"], blk["edge"]

        hn = _tc_node_mlp(x, *_mlp_ws(npar["node_net"]))
        he = _tc_edge_mlp(hb, *_mlp_ws(npar["edge_net"]))
        seg2 = _sc_node_agg(he.reshape(NW * NB, BATCH, NODE_DIM), hn,
                            row3, col3).reshape(2, N_PAD, NODE_DIM)
        x, tl, tr, cbl, cbr = _tc_node_finish(
            x, hn, seg2, deg2,
            npar["msg_net"]["w"], _r1(npar["msg_net"]["b"]),
            npar["centroid_lin"]["w"], _r1(npar["centroid_lin"]["b"]),
            *_mlp_ws(npar["out_layer"]),
            _r1(npar["layer_norm"]["g"]), _r1(npar["layer_norm"]["b"]),
            epar["bond_ffn_left"]["node_linear"]["w"],
            epar["bond_ffn_right"]["node_linear"]["w"],
            epar["node_ffn_left"]["w"], epar["node_ffn_right"]["w"])

        gl3, gr3 = _sc_gather32(tl, tr, row3, col3)
        mbl, mbr = _tc_edge_inter(
            hb, gl3.reshape(N_EDGES, INTER), gr3.reshape(N_EDGES, INTER),
            epar["bond_ffn_left"]["bond_linear"]["w"],
            *_mlp_ws(epar["bond_ffn_left"]["inter_module"]),
            epar["bond_ffn_right"]["bond_linear"]["w"],
            *_mlp_ws(epar["bond_ffn_right"]["inter_module"]))

        sl2, sr2 = _sc_scatter2(mbl.reshape(NW * NB, BATCH, EDGE_DIM),
                                mbr.reshape(NW * NB, BATCH, EDGE_DIM),
                                col3, row3)
        sl2 = sl2.reshape(2, N_PAD, 16)
        sr2 = sr2.reshape(2, N_PAD, 16)
        cL, cR = _tc_node_mix(sl2, sr2, cbl, cbr,
                              epar["msg_left"]["w"], epar["msg_right"]["w"])

        gL3, gR3 = _sc_gather16(cL, cR, row3, col3)
        ball = _r1(epar["msg_left"]["b"] + epar["node_ffn_left"]["b"]
                   + epar["msg_right"]["b"] + epar["node_ffn_right"]["b"]
                   + epar["self_ffn"]["b"])
        hb = _tc_edge_finish(
            gL3.reshape(N_EDGES, EDGE_DIM), gR3.reshape(N_EDGES, EDGE_DIM),
            hb, epar["self_ffn"]["w"], ball,
            *_mlp_ws(epar["out_layer"]),
            _r1(epar["layer_norm"]["g"]), _r1(epar["layer_norm"]["b"]))

    return x, hb


# speedup vs baseline: 3.2445x; 1.2019x over previous
import functools

import jax
import jax.numpy as jnp
from jax import lax
from jax.experimental import pallas as pl
from jax.experimental.pallas import tpu as pltpu
from jax.experimental.pallas import tpu_sc as plsc

N_NODES = 10000
N_EDGES = 160000
NODE_DIM = 128
EDGE_DIM = 16
INTER = 32

NC, NS = 2, 16
NW = NC * NS
BATCH = 125
EPW = N_EDGES // NW
NB = EPW // BATCH
BATCH_A = 50
NB_A = EPW // BATCH_A
N_PAD = 10240
ROWS_PT = N_PAD // NS
ZCH = 40

_MESH = plsc.VectorSubcoreMesh(core_axis_name="c", subcore_axis_name="s")


def _f32(*shape):
    return jax.ShapeDtypeStruct(shape, jnp.float32)


def _fill(ref, rows, width, val):
    def body(r, _):
        for cidx in range(width // 16):
            ref[r, pl.ds(cidx * 16, 16)] = jnp.full((16,), val, jnp.float32)
        return 0
    lax.fori_loop(0, rows, body, 0)


def _worker_id():
    return lax.axis_index("s") * NC + lax.axis_index("c")


def _make_sc_deg():
    def body(row_i, out, iR, buf, zb, acc, sem):
        c = lax.axis_index("c")
        s = lax.axis_index("s")
        w = s * NC + c
        _fill(zb, ZCH, 16, 0.0)
        for z in range(ROWS_PT // ZCH):
            pltpu.sync_copy(zb, acc.at[pl.ds(s * ROWS_PT + z * ZCH, ZCH)])
        plsc.subcore_barrier()
        _fill(buf, BATCH, 16, 1.0)
        pltpu.sync_copy(row_i.at[w], iR)

        def step(j, _):
            pltpu.sync_copy(buf, acc.at[iR.at[j]], add=True)
            return 0
        lax.fori_loop(0, NB, step, 0)
        plsc.subcore_barrier()
        pltpu.sync_copy(acc.at[pl.ds(s * ROWS_PT, ROWS_PT)],
                        out.at[pl.ds(c * N_PAD + s * ROWS_PT, ROWS_PT)])

    return pl.kernel(
        body,
        compiler_params=pltpu.CompilerParams(use_tc_tiling_on_sc=False),
        out_type=_f32(2 * N_PAD, 16),
        mesh=_MESH,
        scratch_types=[
            pltpu.VMEM((NB, BATCH), jnp.int32),
            pltpu.VMEM((BATCH, 16), jnp.float32),
            pltpu.VMEM((ZCH, 16), jnp.float32),
            pltpu.VMEM_SHARED((N_PAD, 16), jnp.float32),
            pltpu.SemaphoreType.DMA,
        ],
    )


def _make_sc_node_agg():
    def body(he3, hn, row_i, col_i, out, rV, cV, bufA0, bufB0, bufA1, bufB1,
             zb, acc, semA0, semB0, semA1, semB1):
        c = lax.axis_index("c")
        s = lax.axis_index("s")
        w = s * NC + c
        _fill(zb, ZCH, NODE_DIM, 0.0)
        for z in range(ROWS_PT // ZCH):
            pltpu.sync_copy(zb, acc.at[pl.ds(s * ROWS_PT + z * ZCH, ZCH)])
        plsc.subcore_barrier()
        pltpu.sync_copy(row_i.at[w], rV)
        pltpu.sync_copy(col_i.at[w], cV)

        def issue(j, bufA, bufB, semA, semB):
            dA = pltpu.async_copy(he3.at[w * NB_A + j], bufA, semA)
            dB = pltpu.async_copy(hn.at[cV.at[j]], bufB, semB)
            return dA, dB

        def drain(j, dA, dB, bufA, bufB):
            dA.wait()
            dB.wait()
            pltpu.sync_copy(bufA, acc.at[rV.at[j]], add=True)
            pltpu.sync_copy(bufB, acc.at[rV.at[j]], add=True)

        d0 = issue(0, bufA0, bufB0, semA0, semB0)

        def pair(i, _):
            j0 = 2 * i
            d1 = issue(j0 + 1, bufA1, bufB1, semA1, semB1)
            drain(j0, *d0, bufA0, bufB0)
            issue(j0 + 2, bufA0, bufB0, semA0, semB0)
            drain(j0 + 1, *d1, bufA1, bufB1)
            return 0
        lax.fori_loop(0, NB_A // 2 - 1, pair, 0)
        d1 = issue(NB_A - 1, bufA1, bufB1, semA1, semB1)
        drain(NB_A - 2, *d0, bufA0, bufB0)
        drain(NB_A - 1, *d1, bufA1, bufB1)
        plsc.subcore_barrier()
        pltpu.sync_copy(acc.at[pl.ds(s * ROWS_PT, ROWS_PT)],
                        out.at[pl.ds(c * N_PAD + s * ROWS_PT, ROWS_PT)])

    return pl.kernel(
        body,
        compiler_params=pltpu.CompilerParams(use_tc_tiling_on_sc=False),
        out_type=_f32(2 * N_PAD, NODE_DIM),
        mesh=_MESH,
        scratch_types=[
            pltpu.VMEM((NB_A, BATCH_A), jnp.int32),
            pltpu.VMEM((NB_A, BATCH_A), jnp.int32),
            pltpu.VMEM((BATCH_A, NODE_DIM), jnp.float32),
            pltpu.VMEM((BATCH_A, NODE_DIM), jnp.float32),
            pltpu.VMEM((BATCH_A, NODE_DIM), jnp.float32),
            pltpu.VMEM((BATCH_A, NODE_DIM), jnp.float32),
            pltpu.VMEM((ZCH, NODE_DIM), jnp.float32),
            pltpu.VMEM_SHARED((N_PAD, NODE_DIM), jnp.float32),
            pltpu.SemaphoreType.DMA,
            pltpu.SemaphoreType.DMA,
            pltpu.SemaphoreType.DMA,
            pltpu.SemaphoreType.DMA,
        ],
    )


def _make_sc_gather2(wa, wb):
    def body(tabA, tabB, idxA, idxB, gA, gB, iA, iB, bufA0, bufB0,
             bufA1, bufB1, semA0, semB0, semA1, semB1):
        c = lax.axis_index("c")
        s = lax.axis_index("s")
        w = s * NC + c
        pltpu.sync_copy(idxA.at[w], iA)
        pltpu.sync_copy(idxB.at[w], iB)

        def issue(j, bufA, bufB, semA, semB):
            dA = pltpu.async_copy(tabA.at[iA.at[j]], bufA, semA)
            dB = pltpu.async_copy(tabB.at[iB.at[j]], bufB, semB)
            return dA, dB

        def drain(j, dA, dB, bufA, bufB):
            dA.wait()
            pltpu.sync_copy(bufA, gA.at[w * NB + j])
            dB.wait()
            pltpu.sync_copy(bufB, gB.at[w * NB + j])

        d0 = issue(0, bufA0, bufB0, semA0, semB0)

        def pair(i, _):
            j0 = 2 * i
            d1 = issue(j0 + 1, bufA1, bufB1, semA1, semB1)
            drain(j0, *d0, bufA0, bufB0)
            issue(j0 + 2, bufA0, bufB0, semA0, semB0)
            drain(j0 + 1, *d1, bufA1, bufB1)
            return 0
        lax.fori_loop(0, NB // 2 - 1, pair, 0)
        d1 = issue(NB - 1, bufA1, bufB1, semA1, semB1)
        drain(NB - 2, *d0, bufA0, bufB0)
        drain(NB - 1, *d1, bufA1, bufB1)

    return pl.kernel(
        body,
        compiler_params=pltpu.CompilerParams(use_tc_tiling_on_sc=False),
        out_type=(_f32(NW * NB, BATCH, wa), _f32(NW * NB, BATCH, wb)),
        mesh=_MESH,
        scratch_types=[
            pltpu.VMEM((NB, BATCH), jnp.int32),
            pltpu.VMEM((NB, BATCH), jnp.int32),
            pltpu.VMEM((BATCH, wa), jnp.float32),
            pltpu.VMEM((BATCH, wb), jnp.float32),
            pltpu.VMEM((BATCH, wa), jnp.float32),
            pltpu.VMEM((BATCH, wb), jnp.float32),
            pltpu.SemaphoreType.DMA,
            pltpu.SemaphoreType.DMA,
            pltpu.SemaphoreType.DMA,
            pltpu.SemaphoreType.DMA,
        ],
    )


def _make_sc_scatter2():
    def body(vA3, vB3, idxA, idxB, oA, oB, iA, iB, bufA0, bufB0,
             bufA1, bufB1, zb, accA, accB, semA0, semB0, semA1, semB1):
        c = lax.axis_index("c")
        s = lax.axis_index("s")
        w = s * NC + c
        _fill(zb, ZCH, 16, 0.0)
        for z in range(ROWS_PT // ZCH):
            pltpu.sync_copy(zb, accA.at[pl.ds(s * ROWS_PT + z * ZCH, ZCH)])
            pltpu.sync_copy(zb, accB.at[pl.ds(s * ROWS_PT + z * ZCH, ZCH)])
        plsc.subcore_barrier()
        pltpu.sync_copy(idxA.at[w], iA)
        pltpu.sync_copy(idxB.at[w], iB)

        def issue(j, bufA, bufB, semA, semB):
            dA = pltpu.async_copy(vA3.at[w * NB + j], bufA, semA)
            dB = pltpu.async_copy(vB3.at[w * NB + j], bufB, semB)
            return dA, dB

        def drain(j, dA, dB, bufA, bufB):
            dA.wait()
            pltpu.sync_copy(bufA, accA.at[iA.at[j]], add=True)
            dB.wait()
            pltpu.sync_copy(bufB, accB.at[iB.at[j]], add=True)

        d0 = issue(0, bufA0, bufB0, semA0, semB0)

        def pair(i, _):
            j0 = 2 * i
            d1 = issue(j0 + 1, bufA1, bufB1, semA1, semB1)
            drain(j0, *d0, bufA0, bufB0)
            issue(j0 + 2, bufA0, bufB0, semA0, semB0)
            drain(j0 + 1, *d1, bufA1, bufB1)
            return 0
        lax.fori_loop(0, NB // 2 - 1, pair, 0)
        d1 = issue(NB - 1, bufA1, bufB1, semA1, semB1)
        drain(NB - 2, *d0, bufA0, bufB0)
        drain(NB - 1, *d1, bufA1, bufB1)
        plsc.subcore_barrier()
        pltpu.sync_copy(accA.at[pl.ds(s * ROWS_PT, ROWS_PT)],
                        oA.at[pl.ds(c * N_PAD + s * ROWS_PT, ROWS_PT)])
        pltpu.sync_copy(accB.at[pl.ds(s * ROWS_PT, ROWS_PT)],
                        oB.at[pl.ds(c * N_PAD + s * ROWS_PT, ROWS_PT)])

    return pl.kernel(
        body,
        compiler_params=pltpu.CompilerParams(use_tc_tiling_on_sc=False),
        out_type=(_f32(2 * N_PAD, 16), _f32(2 * N_PAD, 16)),
        mesh=_MESH,
        scratch_types=[
            pltpu.VMEM((NB, BATCH), jnp.int32),
            pltpu.VMEM((NB, BATCH), jnp.int32),
            pltpu.VMEM((BATCH, 16), jnp.float32),
            pltpu.VMEM((BATCH, 16), jnp.float32),
            pltpu.VMEM((BATCH, 16), jnp.float32),
            pltpu.VMEM((BATCH, 16), jnp.float32),
            pltpu.VMEM((ZCH, 16), jnp.float32),
            pltpu.VMEM_SHARED((N_PAD, 16), jnp.float32),
            pltpu.VMEM_SHARED((N_PAD, 16), jnp.float32),
            pltpu.SemaphoreType.DMA,
            pltpu.SemaphoreType.DMA,
            pltpu.SemaphoreType.DMA,
            pltpu.SemaphoreType.DMA,
        ],
    )


_sc_deg = _make_sc_deg()
_sc_node_agg = _make_sc_node_agg()
_sc_gather32 = _make_sc_gather2(INTER, INTER)
_sc_gather16 = _make_sc_gather2(16, 16)
_sc_scatter2 = _make_sc_scatter2()


def _ln(x, g, b):
    m = jnp.mean(x, axis=-1, keepdims=True)
    v = jnp.mean((x - m) ** 2, axis=-1, keepdims=True)
    return (x - m) * lax.rsqrt(v + 1e-5) * g + b


def _dot(x, w):
    return jnp.dot(x, w, preferred_element_type=jnp.float32)


BN = 5000
BE = 8000


def _wspec(r, c):
    return pl.BlockSpec((r, c), lambda i: (0, 0))


def _rspec(rows, cols):
    return pl.BlockSpec((rows, cols), lambda i: (i, 0))


def _r3spec(rows, cols, half):
    return pl.BlockSpec((1, rows, cols), lambda i, h=half: (h, i, 0))


def _tc_mlp_kernel(x, w0, b0, g, bb, w1, b1, o):
    h = _dot(x[...], w0[...]) + b0[...]
    h = jnp.maximum(_ln(h, g[...], bb[...]), 0.0)
    o[...] = _dot(h, w1[...]) + b1[...]


def _tc_node_mlp(x, w0, b0, g, bb, w1, b1):
    return pl.pallas_call(
        _tc_mlp_kernel,
        grid=(N_NODES // BN,),
        in_specs=[_rspec(BN, NODE_DIM), _wspec(NODE_DIM, NODE_DIM),
                  _wspec(1, NODE_DIM), _wspec(1, NODE_DIM),
                  _wspec(1, NODE_DIM), _wspec(NODE_DIM, NODE_DIM),
                  _wspec(1, NODE_DIM)],
        out_specs=_rspec(BN, NODE_DIM),
        out_shape=_f32(N_NODES, NODE_DIM),
    )(x, w0, b0, g, bb, w1, b1)


def _tc_edge_mlp(x, w0, b0, g, bb, w1, b1):
    return pl.pallas_call(
        _tc_mlp_kernel,
        grid=(N_EDGES // BE,),
        in_specs=[_rspec(BE, EDGE_DIM), _wspec(EDGE_DIM, NODE_DIM),
                  _wspec(1, NODE_DIM), _wspec(1, NODE_DIM),
                  _wspec(1, NODE_DIM), _wspec(NODE_DIM, NODE_DIM),
                  _wspec(1, NODE_DIM)],
        out_specs=_rspec(BE, NODE_DIM),
        out_shape=_f32(N_EDGES, NODE_DIM),
    )(x, w0, b0, g, bb, w1, b1)


def _tc_node_finish_kernel(x, hn, seg_lo, seg_hi, deg_lo, deg_hi,
                           bm, wc, bc, o0, ob0, og, obb, o1, ob1, fg, fb,
                           wtl, wtr, wcbl, wcbr,
                           xn, tl, tr, cbl, cbr):
    seg = seg_lo[0] + seg_hi[0]
    deg = deg_lo[0][:, :1] + deg_hi[0][:, :1]
    xb = x[...]
    aggr = seg + deg * hn[...] + deg * bm[...]
    t = _dot(xb, wc[...]) + bc[...] + aggr
    h = _dot(t, o0[...]) + ob0[...]
    h = jnp.maximum(_ln(h, og[...], obb[...]), 0.0)
    t2 = _dot(h, o1[...]) + ob1[...]
    y = _ln(t2 + xb, fg[...], fb[...])
    xn[...] = y
    tl[...] = _dot(y, wtl[...])
    tr[...] = _dot(y, wtr[...])
    cbl[...] = _dot(y, wcbl[...])
    cbr[...] = _dot(y, wcbr[...])


def _tc_node_finish(x, hn, seg2, deg2, bm, wc, bc, o0, ob0, og, obb,
                    o1, ob1, fg, fb, wtl, wtr, wcbl, wcbr):
    return pl.pallas_call(
        _tc_node_finish_kernel,
        grid=(N_NODES // BN,),
        in_specs=[_rspec(BN, NODE_DIM), _rspec(BN, NODE_DIM),
                  _r3spec(BN, NODE_DIM, 0), _r3spec(BN, NODE_DIM, 1),
                  _r3spec(BN, 16, 0), _r3spec(BN, 16, 1),
                  _wspec(1, NODE_DIM),
                  _wspec(NODE_DIM, NODE_DIM), _wspec(1, NODE_DIM),
                  _wspec(NODE_DIM, NODE_DIM), _wspec(1, NODE_DIM),
                  _wspec(1, NODE_DIM), _wspec(1, NODE_DIM),
                  _wspec(NODE_DIM, NODE_DIM), _wspec(1, NODE_DIM),
                  _wspec(1, NODE_DIM), _wspec(1, NODE_DIM),
                  _wspec(NODE_DIM, INTER), _wspec(NODE_DIM, INTER),
                  _wspec(NODE_DIM, 16), _wspec(NODE_DIM, 16)],
        out_specs=[_rspec(BN, NODE_DIM), _rspec(BN, INTER), _rspec(BN, INTER),
                   _rspec(BN, 16), _rspec(BN, 16)],
        out_shape=(_f32(N_NODES, NODE_DIM), _f32(N_NODES, INTER),
                   _f32(N_NODES, INTER), _f32(N_NODES, 16),
                   _f32(N_NODES, 16)),
    )(x, hn, seg2, seg2, deg2, deg2, bm, wc, bc, o0, ob0, og, obb,
      o1, ob1, fg, fb, wtl, wtr, wcbl, wcbr)


def _tc_edge_inter_kernel(hb, gl, gr, bl, l0, lb0, lg, lbb, l1, lb1,
                          br, r0, rb0, rg, rbb, r1, rb1, mbl, mbr):
    hbb = hb[...]
    il = _dot(hbb, bl[...]) + gl[...]
    h = _dot(il, l0[...]) + lb0[...]
    h = jnp.maximum(_ln(h, lg[...], lbb[...]), 0.0)
    mbl[...] = _dot(h, l1[...]) + lb1[...]
    ir = _dot(hbb, br[...]) + gr[...]
    h = _dot(ir, r0[...]) + rb0[...]
    h = jnp.maximum(_ln(h, rg[...], rbb[...]), 0.0)
    mbr[...] = _dot(h, r1[...]) + rb1[...]


def _tc_edge_inter(hb, gl, gr, *ws):
    return pl.pallas_call(
        _tc_edge_inter_kernel,
        grid=(N_EDGES // BE,),
        in_specs=[_rspec(BE, EDGE_DIM), _rspec(BE, INTER), _rspec(BE, INTER),
                  _wspec(EDGE_DIM, INTER), _wspec(INTER, INTER),
                  _wspec(1, INTER), _wspec(1, INTER), _wspec(1, INTER),
                  _wspec(INTER, EDGE_DIM), _wspec(1, EDGE_DIM),
                  _wspec(EDGE_DIM, INTER), _wspec(INTER, INTER),
                  _wspec(1, INTER), _wspec(1, INTER), _wspec(1, INTER),
                  _wspec(INTER, EDGE_DIM), _wspec(1, EDGE_DIM)],
        out_specs=[_rspec(BE, EDGE_DIM), _rspec(BE, EDGE_DIM)],
        out_shape=(_f32(N_EDGES, EDGE_DIM), _f32(N_EDGES, EDGE_DIM)),
    )(hb, gl, gr, *ws)


def _tc_node_mix_kernel(sl_lo, sl_hi, sr_lo, sr_hi, cbl, cbr, ml, mr,
                        cL, cR):
    cL[...] = _dot(sl_lo[0] + sl_hi[0], ml[...]) + cbl[...]
    cR[...] = _dot(sr_lo[0] + sr_hi[0], mr[...]) + cbr[...]


def _tc_node_mix(sl2, sr2, cbl, cbr, ml, mr):
    return pl.pallas_call(
        _tc_node_mix_kernel,
        grid=(N_NODES // BN,),
        in_specs=[_r3spec(BN, 16, 0), _r3spec(BN, 16, 1),
                  _r3spec(BN, 16, 0), _r3spec(BN, 16, 1),
                  _rspec(BN, 16), _rspec(BN, 16),
                  _wspec(16, 16), _wspec(16, 16)],
        out_specs=[_rspec(BN, 16), _rspec(BN, 16)],
        out_shape=(_f32(N_NODES, 16), _f32(N_NODES, 16)),
    )(sl2, sl2, sr2, sr2, cbl, cbr, ml, mr)


def _tc_edge_finish_kernel(gL, gR, hb, ws, ball, o0, ob0, og, obb, o1, ob1,
                           fg, fb, out):
    hbb = hb[...]
    upd = gL[...] + gR[...] + _dot(hbb, ws[...]) + ball[...]
    h = _dot(upd, o0[...]) + ob0[...]
    h = jnp.maximum(_ln(h, og[...], obb[...]), 0.0)
    t = _dot(h, o1[...]) + ob1[...]
    out[...] = _ln(t + hbb, fg[...], fb[...])


def _tc_edge_finish(gL, gR, hb, *ws):
    return pl.pallas_call(
        _tc_edge_finish_kernel,
        grid=(N_EDGES // BE,),
        in_specs=[_rspec(BE, EDGE_DIM), _rspec(BE, EDGE_DIM),
                  _rspec(BE, EDGE_DIM),
                  _wspec(EDGE_DIM, EDGE_DIM), _wspec(1, EDGE_DIM),
                  _wspec(EDGE_DIM, EDGE_DIM), _wspec(1, EDGE_DIM),
                  _wspec(1, EDGE_DIM), _wspec(1, EDGE_DIM),
                  _wspec(EDGE_DIM, EDGE_DIM), _wspec(1, EDGE_DIM),
                  _wspec(1, EDGE_DIM), _wspec(1, EDGE_DIM)],
        out_specs=_rspec(BE, EDGE_DIM),
        out_shape=_f32(N_EDGES, EDGE_DIM),
    )(gL, gR, hb, *ws)


def _r1(v):
    return v.reshape(1, -1)


def _mlp_ws(p):
    return (p["l0"]["w"], _r1(p["l0"]["b"]), _r1(p["ln"]["g"]),
            _r1(p["ln"]["b"]), p["l1"]["w"], _r1(p["l1"]["b"]))


def _mlp_ws_fold(p, wm):
    return (p["l0"]["w"], _r1(p["l0"]["b"]), _r1(p["ln"]["g"]),
            _r1(p["ln"]["b"]), p["l1"]["w"] @ wm, _r1(p["l1"]["b"] @ wm))


def kernel(h_node, h_edge, edge_index, node_extra, edge_extra, params):
    row3 = edge_index[0].reshape(NW, NB, BATCH)
    col3 = edge_index[1].reshape(NW, NB, BATCH)
    row3a = edge_index[0].reshape(NW, NB_A, BATCH_A)
    col3a = edge_index[1].reshape(NW, NB_A, BATCH_A)

    deg2 = _sc_deg(row3).reshape(2, N_PAD, 16)

    x, hb = h_node, h_edge
    for blk in params["blocks"]:
        npar, epar = blk["node"], blk["edge"]

        wm = npar["msg_net"]["w"]
        hn = _tc_node_mlp(x, *_mlp_ws_fold(npar["node_net"], wm))
        he = _tc_edge_mlp(hb, *_mlp_ws_fold(npar["edge_net"], wm))
        seg2 = _sc_node_agg(he.reshape(NW * NB_A, BATCH_A, NODE_DIM), hn,
                            row3a, col3a).reshape(2, N_PAD, NODE_DIM)
        x, tl, tr, cbl, cbr = _tc_node_finish(
            x, hn, seg2, deg2,
            _r1(npar["msg_net"]["b"]),
            npar["centroid_lin"]["w"], _r1(npar["centroid_lin"]["b"]),
            *_mlp_ws(npar["out_layer"]),
            _r1(npar["layer_norm"]["g"]), _r1(npar["layer_norm"]["b"]),
            epar["bond_ffn_left"]["node_linear"]["w"],
            epar["bond_ffn_right"]["node_linear"]["w"],
            epar["node_ffn_left"]["w"], epar["node_ffn_right"]["w"])

        gl3, gr3 = _sc_gather32(tl, tr, row3, col3)
        mbl, mbr = _tc_edge_inter(
            hb, gl3.reshape(N_EDGES, INTER), gr3.reshape(N_EDGES, INTER),
            epar["bond_ffn_left"]["bond_linear"]["w"],
            *_mlp_ws(epar["bond_ffn_left"]["inter_module"]),
            epar["bond_ffn_right"]["bond_linear"]["w"],
            *_mlp_ws(epar["bond_ffn_right"]["inter_module"]))

        sl2, sr2 = _sc_scatter2(mbl.reshape(NW * NB, BATCH, EDGE_DIM),
                                mbr.reshape(NW * NB, BATCH, EDGE_DIM),
                                col3, row3)
        sl2 = sl2.reshape(2, N_PAD, 16)
        sr2 = sr2.reshape(2, N_PAD, 16)
        cL, cR = _tc_node_mix(sl2, sr2, cbl, cbr,
                              epar["msg_left"]["w"], epar["msg_right"]["w"])

        gL3, gR3 = _sc_gather16(cL, cR, row3, col3)
        ball = _r1(epar["msg_left"]["b"] + epar["node_ffn_left"]["b"]
                   + epar["msg_right"]["b"] + epar["node_ffn_right"]["b"]
                   + epar["self_ffn"]["b"])
        hb = _tc_edge_finish(
            gL3.reshape(N_EDGES, EDGE_DIM), gR3.reshape(N_EDGES, EDGE_DIM),
            hb, epar["self_ffn"]["w"], ball,
            *_mlp_ws(epar["out_layer"]),
            _r1(epar["layer_norm"]["g"]), _r1(epar["layer_norm"]["b"]))

    return x, hb


# speedup vs baseline: 3.3737x; 1.0398x over previous
import functools

import jax
import jax.numpy as jnp
from jax import lax
from jax.experimental import pallas as pl
from jax.experimental.pallas import tpu as pltpu
from jax.experimental.pallas import tpu_sc as plsc

N_NODES = 10000
N_EDGES = 160000
NODE_DIM = 128
EDGE_DIM = 16
INTER = 32

NC, NS = 2, 16
NW = NC * NS
BATCH = 125
EPW = N_EDGES // NW
NB = EPW // BATCH
BATCH_A = 50
NB_A = EPW // BATCH_A
N_PAD = 10240
ROWS_PT = N_PAD // NS
ZCH = 40
EPS = N_EDGES // NS
NBS = EPS // BATCH
ROWS_LAST = N_NODES - (NS - 1) * ROWS_PT

_MESH = plsc.VectorSubcoreMesh(core_axis_name="c", subcore_axis_name="s")


def _f32(*shape):
    return jax.ShapeDtypeStruct(shape, jnp.float32)


def _fill(ref, rows, width, val):
    def body(r, _):
        for cidx in range(width // 16):
            ref[r, pl.ds(cidx * 16, 16)] = jnp.full((16,), val, jnp.float32)
        return 0
    lax.fori_loop(0, rows, body, 0)


def _worker_id():
    return lax.axis_index("s") * NC + lax.axis_index("c")


def _make_sc_deg():
    def body(row_i, out, iR, buf, zb, acc, sem):
        c = lax.axis_index("c")
        s = lax.axis_index("s")
        w = s * NC + c
        _fill(zb, ZCH, 16, 0.0)
        for z in range(ROWS_PT // ZCH):
            pltpu.sync_copy(zb, acc.at[pl.ds(s * ROWS_PT + z * ZCH, ZCH)])
        plsc.subcore_barrier()
        _fill(buf, BATCH, 16, 1.0)
        pltpu.sync_copy(row_i.at[w], iR)

        def step(j, _):
            pltpu.sync_copy(buf, acc.at[iR.at[j]], add=True)
            return 0
        lax.fori_loop(0, NB, step, 0)
        plsc.subcore_barrier()
        pltpu.sync_copy(acc.at[pl.ds(s * ROWS_PT, ROWS_PT)],
                        out.at[pl.ds(c * N_PAD + s * ROWS_PT, ROWS_PT)])

    return pl.kernel(
        body,
        compiler_params=pltpu.CompilerParams(use_tc_tiling_on_sc=False),
        out_type=_f32(2 * N_PAD, 16),
        mesh=_MESH,
        scratch_types=[
            pltpu.VMEM((NB, BATCH), jnp.int32),
            pltpu.VMEM((BATCH, 16), jnp.float32),
            pltpu.VMEM((ZCH, 16), jnp.float32),
            pltpu.VMEM_SHARED((N_PAD, 16), jnp.float32),
            pltpu.SemaphoreType.DMA,
        ],
    )


def _make_sc_node_agg():
    def body(he3, hn, row_i, col_i, out, rV, cV, bufA0, bufB0, bufA1, bufB1,
             zb, acc, semA0, semB0, semA1, semB1):
        c = lax.axis_index("c")
        s = lax.axis_index("s")
        w = s * NC + c
        _fill(zb, ZCH, NODE_DIM, 0.0)
        for z in range(ROWS_PT // ZCH):
            pltpu.sync_copy(zb, acc.at[pl.ds(s * ROWS_PT + z * ZCH, ZCH)])
        plsc.subcore_barrier()
        pltpu.sync_copy(row_i.at[w], rV)
        pltpu.sync_copy(col_i.at[w], cV)

        def issue(j, bufA, bufB, semA, semB):
            dA = pltpu.async_copy(he3.at[w * NB_A + j], bufA, semA)
            dB = pltpu.async_copy(hn.at[cV.at[j]], bufB, semB)
            return dA, dB

        def drain(j, dA, dB, bufA, bufB):
            dA.wait()
            dB.wait()
            pltpu.sync_copy(bufA, acc.at[rV.at[j]], add=True)
            pltpu.sync_copy(bufB, acc.at[rV.at[j]], add=True)

        d0 = issue(0, bufA0, bufB0, semA0, semB0)

        def pair(i, _):
            j0 = 2 * i
            d1 = issue(j0 + 1, bufA1, bufB1, semA1, semB1)
            drain(j0, *d0, bufA0, bufB0)
            issue(j0 + 2, bufA0, bufB0, semA0, semB0)
            drain(j0 + 1, *d1, bufA1, bufB1)
            return 0
        lax.fori_loop(0, NB_A // 2 - 1, pair, 0)
        d1 = issue(NB_A - 1, bufA1, bufB1, semA1, semB1)
        drain(NB_A - 2, *d0, bufA0, bufB0)
        drain(NB_A - 1, *d1, bufA1, bufB1)
        plsc.subcore_barrier()
        pltpu.sync_copy(acc.at[pl.ds(s * ROWS_PT, ROWS_PT)],
                        out.at[pl.ds(c * N_PAD + s * ROWS_PT, ROWS_PT)])

    return pl.kernel(
        body,
        compiler_params=pltpu.CompilerParams(use_tc_tiling_on_sc=False),
        out_type=_f32(2 * N_PAD, NODE_DIM),
        mesh=_MESH,
        scratch_types=[
            pltpu.VMEM((NB_A, BATCH_A), jnp.int32),
            pltpu.VMEM((NB_A, BATCH_A), jnp.int32),
            pltpu.VMEM((BATCH_A, NODE_DIM), jnp.float32),
            pltpu.VMEM((BATCH_A, NODE_DIM), jnp.float32),
            pltpu.VMEM((BATCH_A, NODE_DIM), jnp.float32),
            pltpu.VMEM((BATCH_A, NODE_DIM), jnp.float32),
            pltpu.VMEM((ZCH, NODE_DIM), jnp.float32),
            pltpu.VMEM_SHARED((N_PAD, NODE_DIM), jnp.float32),
            pltpu.SemaphoreType.DMA,
            pltpu.SemaphoreType.DMA,
            pltpu.SemaphoreType.DMA,
            pltpu.SemaphoreType.DMA,
        ],
    )


def _make_sc_gather2(wa, wb):
    def body(tabA, tabB, idxA, idxB, gA, gB, iA, iB, bufA0, bufB0,
             bufA1, bufB1, semA0, semB0, semA1, semB1):
        c = lax.axis_index("c")
        s = lax.axis_index("s")
        w = s * NC + c
        pltpu.sync_copy(idxA.at[w], iA)
        pltpu.sync_copy(idxB.at[w], iB)

        def issue(j, bufA, bufB, semA, semB):
            dA = pltpu.async_copy(tabA.at[iA.at[j]], bufA, semA)
            dB = pltpu.async_copy(tabB.at[iB.at[j]], bufB, semB)
            return dA, dB

        def drain(j, dA, dB, bufA, bufB):
            dA.wait()
            pltpu.sync_copy(bufA, gA.at[w * NB + j])
            dB.wait()
            pltpu.sync_copy(bufB, gB.at[w * NB + j])

        d0 = issue(0, bufA0, bufB0, semA0, semB0)

        def pair(i, _):
            j0 = 2 * i
            d1 = issue(j0 + 1, bufA1, bufB1, semA1, semB1)
            drain(j0, *d0, bufA0, bufB0)
            issue(j0 + 2, bufA0, bufB0, semA0, semB0)
            drain(j0 + 1, *d1, bufA1, bufB1)
            return 0
        lax.fori_loop(0, NB // 2 - 1, pair, 0)
        d1 = issue(NB - 1, bufA1, bufB1, semA1, semB1)
        drain(NB - 2, *d0, bufA0, bufB0)
        drain(NB - 1, *d1, bufA1, bufB1)

    return pl.kernel(
        body,
        compiler_params=pltpu.CompilerParams(use_tc_tiling_on_sc=False),
        out_type=(_f32(NW * NB, BATCH, wa), _f32(NW * NB, BATCH, wb)),
        mesh=_MESH,
        scratch_types=[
            pltpu.VMEM((NB, BATCH), jnp.int32),
            pltpu.VMEM((NB, BATCH), jnp.int32),
            pltpu.VMEM((BATCH, wa), jnp.float32),
            pltpu.VMEM((BATCH, wb), jnp.float32),
            pltpu.VMEM((BATCH, wa), jnp.float32),
            pltpu.VMEM((BATCH, wb), jnp.float32),
            pltpu.SemaphoreType.DMA,
            pltpu.SemaphoreType.DMA,
            pltpu.SemaphoreType.DMA,
            pltpu.SemaphoreType.DMA,
        ],
    )


def _make_sc_scatter_gather():
    def body(vA3, vB3, cbl, cbr, isA, isB, igA, igB, gA, gB,
             iS, iG, buf0, buf1, acc, sem0, sem1):
        c = lax.axis_index("c")
        s = lax.axis_index("s")

        def run(v3, cb, iS_h, iG_h, gout):
            pltpu.sync_copy(iS_h.at[s], iS)
            pltpu.sync_copy(iG_h.at[s], iG)

            @pl.when(s < NS - 1)
            def _():
                pltpu.sync_copy(cb.at[pl.ds(s * ROWS_PT, ROWS_PT)],
                                acc.at[pl.ds(s * ROWS_PT, ROWS_PT)])

            @pl.when(s == NS - 1)
            def _():
                pltpu.sync_copy(
                    cb.at[pl.ds((NS - 1) * ROWS_PT, ROWS_LAST)],
                    acc.at[pl.ds((NS - 1) * ROWS_PT, ROWS_LAST)])
            plsc.subcore_barrier()

            def issue(j, buf, sem):
                return pltpu.async_copy(v3.at[s * NBS + j], buf, sem)

            def drain(j, d, buf):
                d.wait()
                pltpu.sync_copy(buf, acc.at[iS.at[j]], add=True)

            d0 = issue(0, buf0, sem0)

            def pair(i, _):
                j0 = 2 * i
                d1 = issue(j0 + 1, buf1, sem1)
                drain(j0, d0, buf0)
                issue(j0 + 2, buf0, sem0)
                drain(j0 + 1, d1, buf1)
                return 0
            lax.fori_loop(0, NBS // 2 - 1, pair, 0)
            d1 = issue(NBS - 1, buf1, sem1)
            drain(NBS - 2, d0, buf0)
            drain(NBS - 1, d1, buf1)
            plsc.subcore_barrier()

            def gissue(j, buf, sem):
                return pltpu.async_copy(acc.at[iG.at[j]], buf, sem)

            def gdrain(j, d, buf):
                d.wait()
                pltpu.sync_copy(buf, gout.at[s * NBS + j])

            g0 = gissue(0, buf0, sem0)

            def gpair(i, _):
                j0 = 2 * i
                g1 = gissue(j0 + 1, buf1, sem1)
                gdrain(j0, g0, buf0)
                gissue(j0 + 2, buf0, sem0)
                gdrain(j0 + 1, g1, buf1)
                return 0
            lax.fori_loop(0, NBS // 2 - 1, gpair, 0)
            g1 = gissue(NBS - 1, buf1, sem1)
            gdrain(NBS - 2, g0, buf0)
            gdrain(NBS - 1, g1, buf1)

        @pl.when(c == 0)
        def _():
            run(vA3, cbl, isA, igA, gA)

        @pl.when(c == 1)
        def _():
            run(vB3, cbr, isB, igB, gB)

    return pl.kernel(
        body,
        compiler_params=pltpu.CompilerParams(use_tc_tiling_on_sc=False),
        out_type=(_f32(NS * NBS, BATCH, EDGE_DIM),
                  _f32(NS * NBS, BATCH, EDGE_DIM)),
        mesh=_MESH,
        scratch_types=[
            pltpu.VMEM((NBS, BATCH), jnp.int32),
            pltpu.VMEM((NBS, BATCH), jnp.int32),
            pltpu.VMEM((BATCH, EDGE_DIM), jnp.float32),
            pltpu.VMEM((BATCH, EDGE_DIM), jnp.float32),
            pltpu.VMEM_SHARED((N_PAD, EDGE_DIM), jnp.float32),
            pltpu.SemaphoreType.DMA,
            pltpu.SemaphoreType.DMA,
        ],
    )


_sc_deg = _make_sc_deg()
_sc_node_agg = _make_sc_node_agg()
_sc_gather32 = _make_sc_gather2(INTER, INTER)
_sc_scatter_gather = _make_sc_scatter_gather()


def _ln(x, g, b):
    m = jnp.mean(x, axis=-1, keepdims=True)
    v = jnp.mean((x - m) ** 2, axis=-1, keepdims=True)
    return (x - m) * lax.rsqrt(v + 1e-5) * g + b


def _dot(x, w):
    return jnp.dot(x, w, preferred_element_type=jnp.float32)


BN = 5000
BE = 8000


def _wspec(r, c):
    return pl.BlockSpec((r, c), lambda i: (0, 0))


def _rspec(rows, cols):
    return pl.BlockSpec((rows, cols), lambda i: (i, 0))


def _r3spec(rows, cols, half):
    return pl.BlockSpec((1, rows, cols), lambda i, h=half: (h, i, 0))


def _tc_mlp_kernel(x, w0, b0, g, bb, w1, b1, o):
    h = _dot(x[...], w0[...]) + b0[...]
    h = jnp.maximum(_ln(h, g[...], bb[...]), 0.0)
    o[...] = _dot(h, w1[...]) + b1[...]


def _tc_node_mlp(x, w0, b0, g, bb, w1, b1):
    return pl.pallas_call(
        _tc_mlp_kernel,
        grid=(N_NODES // BN,),
        in_specs=[_rspec(BN, NODE_DIM), _wspec(NODE_DIM, NODE_DIM),
                  _wspec(1, NODE_DIM), _wspec(1, NODE_DIM),
                  _wspec(1, NODE_DIM), _wspec(NODE_DIM, NODE_DIM),
                  _wspec(1, NODE_DIM)],
        out_specs=_rspec(BN, NODE_DIM),
        out_shape=_f32(N_NODES, NODE_DIM),
    )(x, w0, b0, g, bb, w1, b1)


def _tc_edge_mlp(x, w0, b0, g, bb, w1, b1):
    return pl.pallas_call(
        _tc_mlp_kernel,
        grid=(N_EDGES // BE,),
        in_specs=[_rspec(BE, EDGE_DIM), _wspec(EDGE_DIM, NODE_DIM),
                  _wspec(1, NODE_DIM), _wspec(1, NODE_DIM),
                  _wspec(1, NODE_DIM), _wspec(NODE_DIM, NODE_DIM),
                  _wspec(1, NODE_DIM)],
        out_specs=_rspec(BE, NODE_DIM),
        out_shape=_f32(N_EDGES, NODE_DIM),
    )(x, w0, b0, g, bb, w1, b1)


def _tc_node_finish_kernel(x, hn, seg_lo, seg_hi, deg_lo, deg_hi,
                           bm, wc, bc, o0, ob0, og, obb, o1, ob1, fg, fb,
                           wtl, wtr, wcbl, wcbr,
                           xn, tl, tr, cbl, cbr):
    seg = seg_lo[0] + seg_hi[0]
    deg = deg_lo[0][:, :1] + deg_hi[0][:, :1]
    xb = x[...]
    aggr = seg + deg * hn[...] + deg * bm[...]
    t = _dot(xb, wc[...]) + bc[...] + aggr
    h = _dot(t, o0[...]) + ob0[...]
    h = jnp.maximum(_ln(h, og[...], obb[...]), 0.0)
    t2 = _dot(h, o1[...]) + ob1[...]
    y = _ln(t2 + xb, fg[...], fb[...])
    xn[...] = y
    tl[...] = _dot(y, wtl[...])
    tr[...] = _dot(y, wtr[...])
    cbl[...] = _dot(y, wcbl[...])
    cbr[...] = _dot(y, wcbr[...])


def _tc_node_finish(x, hn, seg2, deg2, bm, wc, bc, o0, ob0, og, obb,
                    o1, ob1, fg, fb, wtl, wtr, wcbl, wcbr):
    return pl.pallas_call(
        _tc_node_finish_kernel,
        grid=(N_NODES // BN,),
        in_specs=[_rspec(BN, NODE_DIM), _rspec(BN, NODE_DIM),
                  _r3spec(BN, NODE_DIM, 0), _r3spec(BN, NODE_DIM, 1),
                  _r3spec(BN, 16, 0), _r3spec(BN, 16, 1),
                  _wspec(1, NODE_DIM),
                  _wspec(NODE_DIM, NODE_DIM), _wspec(1, NODE_DIM),
                  _wspec(NODE_DIM, NODE_DIM), _wspec(1, NODE_DIM),
                  _wspec(1, NODE_DIM), _wspec(1, NODE_DIM),
                  _wspec(NODE_DIM, NODE_DIM), _wspec(1, NODE_DIM),
                  _wspec(1, NODE_DIM), _wspec(1, NODE_DIM),
                  _wspec(NODE_DIM, INTER), _wspec(NODE_DIM, INTER),
                  _wspec(NODE_DIM, 16), _wspec(NODE_DIM, 16)],
        out_specs=[_rspec(BN, NODE_DIM), _rspec(BN, INTER), _rspec(BN, INTER),
                   _rspec(BN, 16), _rspec(BN, 16)],
        out_shape=(_f32(N_NODES, NODE_DIM), _f32(N_NODES, INTER),
                   _f32(N_NODES, INTER), _f32(N_NODES, 16),
                   _f32(N_NODES, 16)),
    )(x, hn, seg2, seg2, deg2, deg2, bm, wc, bc, o0, ob0, og, obb,
      o1, ob1, fg, fb, wtl, wtr, wcbl, wcbr)


def _tc_edge_inter_kernel(hb, gl, gr, bl, l0, lb0, lg, lbb, l1, lb1,
                          br, r0, rb0, rg, rbb, r1, rb1, mbl, mbr):
    hbb = hb[...]
    il = _dot(hbb, bl[...]) + gl[...]
    h = _dot(il, l0[...]) + lb0[...]
    h = jnp.maximum(_ln(h, lg[...], lbb[...]), 0.0)
    mbl[...] = _dot(h, l1[...]) + lb1[...]
    ir = _dot(hbb, br[...]) + gr[...]
    h = _dot(ir, r0[...]) + rb0[...]
    h = jnp.maximum(_ln(h, rg[...], rbb[...]), 0.0)
    mbr[...] = _dot(h, r1[...]) + rb1[...]


def _tc_edge_inter(hb, gl, gr, *ws):
    return pl.pallas_call(
        _tc_edge_inter_kernel,
        grid=(N_EDGES // BE,),
        in_specs=[_rspec(BE, EDGE_DIM), _rspec(BE, INTER), _rspec(BE, INTER),
                  _wspec(EDGE_DIM, INTER), _wspec(INTER, INTER),
                  _wspec(1, INTER), _wspec(1, INTER), _wspec(1, INTER),
                  _wspec(INTER, EDGE_DIM), _wspec(1, EDGE_DIM),
                  _wspec(EDGE_DIM, INTER), _wspec(INTER, INTER),
                  _wspec(1, INTER), _wspec(1, INTER), _wspec(1, INTER),
                  _wspec(INTER, EDGE_DIM), _wspec(1, EDGE_DIM)],
        out_specs=[_rspec(BE, EDGE_DIM), _rspec(BE, EDGE_DIM)],
        out_shape=(_f32(N_EDGES, EDGE_DIM), _f32(N_EDGES, EDGE_DIM)),
    )(hb, gl, gr, *ws)


def _tc_edge_finish_kernel(gL, gR, hb, ws, ball, o0, ob0, og, obb, o1, ob1,
                           fg, fb, out):
    hbb = hb[...]
    upd = gL[...] + gR[...] + _dot(hbb, ws[...]) + ball[...]
    h = _dot(upd, o0[...]) + ob0[...]
    h = jnp.maximum(_ln(h, og[...], obb[...]), 0.0)
    t = _dot(h, o1[...]) + ob1[...]
    out[...] = _ln(t + hbb, fg[...], fb[...])


def _tc_edge_finish(gL, gR, hb, *ws):
    return pl.pallas_call(
        _tc_edge_finish_kernel,
        grid=(N_EDGES // BE,),
        in_specs=[_rspec(BE, EDGE_DIM), _rspec(BE, EDGE_DIM),
                  _rspec(BE, EDGE_DIM),
                  _wspec(EDGE_DIM, EDGE_DIM), _wspec(1, EDGE_DIM),
                  _wspec(EDGE_DIM, EDGE_DIM), _wspec(1, EDGE_DIM),
                  _wspec(1, EDGE_DIM), _wspec(1, EDGE_DIM),
                  _wspec(EDGE_DIM, EDGE_DIM), _wspec(1, EDGE_DIM),
                  _wspec(1, EDGE_DIM), _wspec(1, EDGE_DIM)],
        out_specs=_rspec(BE, EDGE_DIM),
        out_shape=_f32(N_EDGES, EDGE_DIM),
    )(gL, gR, hb, *ws)


def _r1(v):
    return v.reshape(1, -1)


def _mlp_ws(p):
    return (p["l0"]["w"], _r1(p["l0"]["b"]), _r1(p["ln"]["g"]),
            _r1(p["ln"]["b"]), p["l1"]["w"], _r1(p["l1"]["b"]))


def _mlp_ws_fold(p, wm):
    return (p["l0"]["w"], _r1(p["l0"]["b"]), _r1(p["ln"]["g"]),
            _r1(p["ln"]["b"]), p["l1"]["w"] @ wm, _r1(p["l1"]["b"] @ wm))


def kernel(h_node, h_edge, edge_index, node_extra, edge_extra, params):
    row3 = edge_index[0].reshape(NW, NB, BATCH)
    col3 = edge_index[1].reshape(NW, NB, BATCH)
    row3a = edge_index[0].reshape(NW, NB_A, BATCH_A)
    col3a = edge_index[1].reshape(NW, NB_A, BATCH_A)
    row16 = edge_index[0].reshape(NS, NBS, BATCH)
    col16 = edge_index[1].reshape(NS, NBS, BATCH)

    deg2 = _sc_deg(row3).reshape(2, N_PAD, 16)

    x, hb = h_node, h_edge
    for blk in params["blocks"]:
        npar, epar = blk["node"], blk["edge"]

        wm = npar["msg_net"]["w"]
        hn = _tc_node_mlp(x, *_mlp_ws_fold(npar["node_net"], wm))
        he = _tc_edge_mlp(hb, *_mlp_ws_fold(npar["edge_net"], wm))
        seg2 = _sc_node_agg(he.reshape(NW * NB_A, BATCH_A, NODE_DIM), hn,
                            row3a, col3a).reshape(2, N_PAD, NODE_DIM)
        x, tl, tr, cbl, cbr = _tc_node_finish(
            x, hn, seg2, deg2,
            _r1(npar["msg_net"]["b"]),
            npar["centroid_lin"]["w"], _r1(npar["centroid_lin"]["b"]),
            *_mlp_ws(npar["out_layer"]),
            _r1(npar["layer_norm"]["g"]), _r1(npar["layer_norm"]["b"]),
            epar["bond_ffn_left"]["node_linear"]["w"],
            epar["bond_ffn_right"]["node_linear"]["w"],
            epar["node_ffn_left"]["w"], epar["node_ffn_right"]["w"])

        gl3, gr3 = _sc_gather32(tl, tr, row3, col3)
        mbl, mbr = _tc_edge_inter(
            hb, gl3.reshape(N_EDGES, INTER), gr3.reshape(N_EDGES, INTER),
            epar["bond_ffn_left"]["bond_linear"]["w"],
            *_mlp_ws_fold(epar["bond_ffn_left"]["inter_module"],
                          epar["msg_left"]["w"]),
            epar["bond_ffn_right"]["bond_linear"]["w"],
            *_mlp_ws_fold(epar["bond_ffn_right"]["inter_module"],
                          epar["msg_right"]["w"]))

        gL3, gR3 = _sc_scatter_gather(
            mbl.reshape(NS * NBS, BATCH, EDGE_DIM),
            mbr.reshape(NS * NBS, BATCH, EDGE_DIM),
            cbl, cbr, col16, row16, row16, col16)
        ball = _r1(epar["msg_left"]["b"] + epar["node_ffn_left"]["b"]
                   + epar["msg_right"]["b"] + epar["node_ffn_right"]["b"]
                   + epar["self_ffn"]["b"])
        hb = _tc_edge_finish(
            gL3.reshape(N_EDGES, EDGE_DIM), gR3.reshape(N_EDGES, EDGE_DIM),
            hb, epar["self_ffn"]["w"], ball,
            *_mlp_ws(epar["out_layer"]),
            _r1(epar["layer_norm"]["g"]), _r1(epar["layer_norm"]["b"]))

    return x, hb


# speedup vs baseline: 5.8287x; 1.7277x over previous
import functools

import jax
import jax.numpy as jnp
from jax import lax
from jax.experimental import pallas as pl
from jax.experimental.pallas import tpu as pltpu
from jax.experimental.pallas import tpu_sc as plsc

N_NODES = 10000
N_EDGES = 160000
NODE_DIM = 128
EDGE_DIM = 16
INTER = 32

NC, NS = 2, 16
NW = NC * NS
BATCH = 125
EPW = N_EDGES // NW
NB = EPW // BATCH
BATCH_A = 50
NB_A = EPW // BATCH_A
N_PAD = 10240
ROWS_PT = N_PAD // NS
ZCH = 40
EPS = N_EDGES // NS
NBS = EPS // BATCH
ROWS_LAST = N_NODES - (NS - 1) * ROWS_PT

_MESH = plsc.VectorSubcoreMesh(core_axis_name="c", subcore_axis_name="s")


def _f32(*shape):
    return jax.ShapeDtypeStruct(shape, jnp.float32)


def _fill(ref, rows, width, val):
    def body(r, _):
        for cidx in range(width // 16):
            ref[r, pl.ds(cidx * 16, 16)] = jnp.full((16,), val, jnp.float32)
        return 0
    lax.fori_loop(0, rows, body, 0)


def _worker_id():
    return lax.axis_index("s") * NC + lax.axis_index("c")


def _make_sc_deg():
    def body(row_i, out, iR, buf, zb, acc, sem):
        c = lax.axis_index("c")
        s = lax.axis_index("s")
        w = s * NC + c
        _fill(zb, ZCH, 16, 0.0)
        for z in range(ROWS_PT // ZCH):
            pltpu.sync_copy(zb, acc.at[pl.ds(s * ROWS_PT + z * ZCH, ZCH)])
        plsc.subcore_barrier()
        _fill(buf, BATCH, 16, 1.0)
        pltpu.sync_copy(row_i.at[w], iR)

        def step(j, _):
            pltpu.sync_copy(buf, acc.at[iR.at[j]], add=True)
            return 0
        lax.fori_loop(0, NB, step, 0)
        plsc.subcore_barrier()
        pltpu.sync_copy(acc.at[pl.ds(s * ROWS_PT, ROWS_PT)],
                        out.at[pl.ds(c * N_PAD + s * ROWS_PT, ROWS_PT)])

    return pl.kernel(
        body,
        compiler_params=pltpu.CompilerParams(use_tc_tiling_on_sc=False),
        out_type=_f32(2 * N_PAD, 16),
        mesh=_MESH,
        scratch_types=[
            pltpu.VMEM((NB, BATCH), jnp.int32),
            pltpu.VMEM((BATCH, 16), jnp.float32),
            pltpu.VMEM((ZCH, 16), jnp.float32),
            pltpu.VMEM_SHARED((N_PAD, 16), jnp.float32),
            pltpu.SemaphoreType.DMA,
        ],
    )


def _make_sc_node_agg():
    def body(he3, hn, row_i, col_i, out, rV, cV, bufA0, bufB0, bufA1, bufB1,
             zb, acc, semA0, semB0, semA1, semB1):
        c = lax.axis_index("c")
        s = lax.axis_index("s")
        w = s * NC + c
        _fill(zb, ZCH, NODE_DIM, 0.0)
        for z in range(ROWS_PT // ZCH):
            pltpu.sync_copy(zb, acc.at[pl.ds(s * ROWS_PT + z * ZCH, ZCH)])
        plsc.subcore_barrier()
        pltpu.sync_copy(row_i.at[w], rV)
        pltpu.sync_copy(col_i.at[w], cV)

        def issue(j, bufA, bufB, semA, semB):
            dA = pltpu.async_copy(he3.at[w * NB_A + j], bufA, semA)
            dB = pltpu.async_copy(hn.at[cV.at[j]], bufB, semB)
            return dA, dB

        def drain(j, dA, dB, bufA, bufB):
            dA.wait()
            dB.wait()
            pltpu.sync_copy(bufA, acc.at[rV.at[j]], add=True)
            pltpu.sync_copy(bufB, acc.at[rV.at[j]], add=True)

        d0 = issue(0, bufA0, bufB0, semA0, semB0)

        def pair(i, _):
            j0 = 2 * i
            d1 = issue(j0 + 1, bufA1, bufB1, semA1, semB1)
            drain(j0, *d0, bufA0, bufB0)
            issue(j0 + 2, bufA0, bufB0, semA0, semB0)
            drain(j0 + 1, *d1, bufA1, bufB1)
            return 0
        lax.fori_loop(0, NB_A // 2 - 1, pair, 0)
        d1 = issue(NB_A - 1, bufA1, bufB1, semA1, semB1)
        drain(NB_A - 2, *d0, bufA0, bufB0)
        drain(NB_A - 1, *d1, bufA1, bufB1)
        plsc.subcore_barrier()
        pltpu.sync_copy(acc.at[pl.ds(s * ROWS_PT, ROWS_PT)],
                        out.at[pl.ds(c * N_PAD + s * ROWS_PT, ROWS_PT)])

    return pl.kernel(
        body,
        compiler_params=pltpu.CompilerParams(use_tc_tiling_on_sc=False),
        out_type=_f32(2 * N_PAD, NODE_DIM),
        mesh=_MESH,
        scratch_types=[
            pltpu.VMEM((NB_A, BATCH_A), jnp.int32),
            pltpu.VMEM((NB_A, BATCH_A), jnp.int32),
            pltpu.VMEM((BATCH_A, NODE_DIM), jnp.float32),
            pltpu.VMEM((BATCH_A, NODE_DIM), jnp.float32),
            pltpu.VMEM((BATCH_A, NODE_DIM), jnp.float32),
            pltpu.VMEM((BATCH_A, NODE_DIM), jnp.float32),
            pltpu.VMEM((ZCH, NODE_DIM), jnp.float32),
            pltpu.VMEM_SHARED((N_PAD, NODE_DIM), jnp.float32),
            pltpu.SemaphoreType.DMA,
            pltpu.SemaphoreType.DMA,
            pltpu.SemaphoreType.DMA,
            pltpu.SemaphoreType.DMA,
        ],
    )


def _make_sc_gather2(wa, wb):
    def body(tabA, tabB, idxA, idxB, gA, gB, iA, iB, bufA0, bufB0,
             bufA1, bufB1, semA0, semB0, semA1, semB1):
        c = lax.axis_index("c")
        s = lax.axis_index("s")
        w = s * NC + c
        pltpu.sync_copy(idxA.at[w], iA)
        pltpu.sync_copy(idxB.at[w], iB)

        def issue(j, bufA, bufB, semA, semB):
            dA = pltpu.async_copy(tabA.at[iA.at[j]], bufA, semA)
            dB = pltpu.async_copy(tabB.at[iB.at[j]], bufB, semB)
            return dA, dB

        def drain(j, dA, dB, bufA, bufB):
            dA.wait()
            pltpu.sync_copy(bufA, gA.at[w * NB + j])
            dB.wait()
            pltpu.sync_copy(bufB, gB.at[w * NB + j])

        d0 = issue(0, bufA0, bufB0, semA0, semB0)

        def pair(i, _):
            j0 = 2 * i
            d1 = issue(j0 + 1, bufA1, bufB1, semA1, semB1)
            drain(j0, *d0, bufA0, bufB0)
            issue(j0 + 2, bufA0, bufB0, semA0, semB0)
            drain(j0 + 1, *d1, bufA1, bufB1)
            return 0
        lax.fori_loop(0, NB // 2 - 1, pair, 0)
        d1 = issue(NB - 1, bufA1, bufB1, semA1, semB1)
        drain(NB - 2, *d0, bufA0, bufB0)
        drain(NB - 1, *d1, bufA1, bufB1)

    return pl.kernel(
        body,
        compiler_params=pltpu.CompilerParams(use_tc_tiling_on_sc=False),
        out_type=(_f32(NW * NB, BATCH, wa), _f32(NW * NB, BATCH, wb)),
        mesh=_MESH,
        scratch_types=[
            pltpu.VMEM((NB, BATCH), jnp.int32),
            pltpu.VMEM((NB, BATCH), jnp.int32),
            pltpu.VMEM((BATCH, wa), jnp.float32),
            pltpu.VMEM((BATCH, wb), jnp.float32),
            pltpu.VMEM((BATCH, wa), jnp.float32),
            pltpu.VMEM((BATCH, wb), jnp.float32),
            pltpu.SemaphoreType.DMA,
            pltpu.SemaphoreType.DMA,
            pltpu.SemaphoreType.DMA,
            pltpu.SemaphoreType.DMA,
        ],
    )


def _make_sc_scatter_gather():
    def body(vA3, vB3, cbl, cbr, isA, isB, igA, igB, gA, gB,
             iS, iG, buf0, buf1, acc, sem0, sem1):
        c = lax.axis_index("c")
        s = lax.axis_index("s")

        def run(v3, cb, iS_h, iG_h, gout):
            pltpu.sync_copy(iS_h.at[s], iS)
            pltpu.sync_copy(iG_h.at[s], iG)

            @pl.when(s < NS - 1)
            def _():
                pltpu.sync_copy(cb.at[pl.ds(s * ROWS_PT, ROWS_PT)],
                                acc.at[pl.ds(s * ROWS_PT, ROWS_PT)])

            @pl.when(s == NS - 1)
            def _():
                pltpu.sync_copy(
                    cb.at[pl.ds((NS - 1) * ROWS_PT, ROWS_LAST)],
                    acc.at[pl.ds((NS - 1) * ROWS_PT, ROWS_LAST)])
            plsc.subcore_barrier()

            def issue(j, buf, sem):
                return pltpu.async_copy(v3.at[s * NBS + j], buf, sem)

            def drain(j, d, buf):
                d.wait()
                pltpu.sync_copy(buf, acc.at[iS.at[j]], add=True)

            d0 = issue(0, buf0, sem0)

            def pair(i, _):
                j0 = 2 * i
                d1 = issue(j0 + 1, buf1, sem1)
                drain(j0, d0, buf0)
                issue(j0 + 2, buf0, sem0)
                drain(j0 + 1, d1, buf1)
                return 0
            lax.fori_loop(0, NBS // 2 - 1, pair, 0)
            d1 = issue(NBS - 1, buf1, sem1)
            drain(NBS - 2, d0, buf0)
            drain(NBS - 1, d1, buf1)
            plsc.subcore_barrier()

            def gissue(j, buf, sem):
                return pltpu.async_copy(acc.at[iG.at[j]], buf, sem)

            def gdrain(j, d, buf):
                d.wait()
                pltpu.sync_copy(buf, gout.at[s * NBS + j])

            g0 = gissue(0, buf0, sem0)

            def gpair(i, _):
                j0 = 2 * i
                g1 = gissue(j0 + 1, buf1, sem1)
                gdrain(j0, g0, buf0)
                gissue(j0 + 2, buf0, sem0)
                gdrain(j0 + 1, g1, buf1)
                return 0
            lax.fori_loop(0, NBS // 2 - 1, gpair, 0)
            g1 = gissue(NBS - 1, buf1, sem1)
            gdrain(NBS - 2, g0, buf0)
            gdrain(NBS - 1, g1, buf1)

        @pl.when(c == 0)
        def _():
            run(vA3, cbl, isA, igA, gA)

        @pl.when(c == 1)
        def _():
            run(vB3, cbr, isB, igB, gB)

    return pl.kernel(
        body,
        compiler_params=pltpu.CompilerParams(use_tc_tiling_on_sc=False),
        out_type=(_f32(NS * NBS, BATCH, EDGE_DIM),
                  _f32(NS * NBS, BATCH, EDGE_DIM)),
        mesh=_MESH,
        scratch_types=[
            pltpu.VMEM((NBS, BATCH), jnp.int32),
            pltpu.VMEM((NBS, BATCH), jnp.int32),
            pltpu.VMEM((BATCH, EDGE_DIM), jnp.float32),
            pltpu.VMEM((BATCH, EDGE_DIM), jnp.float32),
            pltpu.VMEM_SHARED((N_PAD, EDGE_DIM), jnp.float32),
            pltpu.SemaphoreType.DMA,
            pltpu.SemaphoreType.DMA,
        ],
    )


_sc_deg = _make_sc_deg()
_sc_node_agg = _make_sc_node_agg()
_sc_gather32 = _make_sc_gather2(INTER, INTER)
_sc_scatter_gather = _make_sc_scatter_gather()


def _ln(x, g, b):
    m = jnp.mean(x, axis=-1, keepdims=True)
    v = jnp.mean((x - m) ** 2, axis=-1, keepdims=True)
    return (x - m) * lax.rsqrt(v + 1e-5) * g + b


def _dot(x, w):
    return jnp.dot(x, w, preferred_element_type=jnp.float32)


BN = 5000
BE = 8000


def _wspec(r, c):
    return pl.BlockSpec((r, c), lambda i: (0, 0))


def _rspec(rows, cols):
    return pl.BlockSpec((rows, cols), lambda i: (i, 0))


def _r3spec(rows, cols, half):
    return pl.BlockSpec((1, rows, cols), lambda i, h=half: (h, i, 0))


def _tc_mlp_kernel(x, w0, b0, g, bb, w1, b1, o):
    h = _dot(x[...], w0[...]) + b0[...]
    h = jnp.maximum(_ln(h, g[...], bb[...]), 0.0)
    o[...] = _dot(h, w1[...]) + b1[...]


def _tc_node_mlp(x, w0, b0, g, bb, w1, b1):
    return pl.pallas_call(
        _tc_mlp_kernel,
        grid=(N_NODES // BN,),
        in_specs=[_rspec(BN, NODE_DIM), _wspec(NODE_DIM, NODE_DIM),
                  _wspec(1, NODE_DIM), _wspec(1, NODE_DIM),
                  _wspec(1, NODE_DIM), _wspec(NODE_DIM, NODE_DIM),
                  _wspec(1, NODE_DIM)],
        out_specs=_rspec(BN, NODE_DIM),
        out_shape=_f32(N_NODES, NODE_DIM),
    )(x, w0, b0, g, bb, w1, b1)


def _tc_edge_mlp(x, w0, b0, g, bb, w1, b1):
    return pl.pallas_call(
        _tc_mlp_kernel,
        grid=(N_EDGES // BE,),
        in_specs=[_rspec(BE, EDGE_DIM), _wspec(EDGE_DIM, NODE_DIM),
                  _wspec(1, NODE_DIM), _wspec(1, NODE_DIM),
                  _wspec(1, NODE_DIM), _wspec(NODE_DIM, NODE_DIM),
                  _wspec(1, NODE_DIM)],
        out_specs=_rspec(BE, NODE_DIM),
        out_shape=_f32(N_EDGES, NODE_DIM),
    )(x, w0, b0, g, bb, w1, b1)


def _tc_node_finish_kernel(x, hn, seg_lo, seg_hi, deg_lo, deg_hi,
                           bm, wc, bc, o0, ob0, og, obb, o1, ob1, fg, fb,
                           wtl, wtr, wcbl, wcbr,
                           xn, tl, tr, cbl, cbr):
    seg = seg_lo[0] + seg_hi[0]
    deg = deg_lo[0][:, :1] + deg_hi[0][:, :1]
    xb = x[...]
    aggr = seg + deg * hn[...] + deg * bm[...]
    t = _dot(xb, wc[...]) + bc[...] + aggr
    h = _dot(t, o0[...]) + ob0[...]
    h = jnp.maximum(_ln(h, og[...], obb[...]), 0.0)
    t2 = _dot(h, o1[...]) + ob1[...]
    y = _ln(t2 + xb, fg[...], fb[...])
    xn[...] = y
    tl[...] = _dot(y, wtl[...])
    tr[...] = _dot(y, wtr[...])
    cbl[...] = _dot(y, wcbl[...])
    cbr[...] = _dot(y, wcbr[...])


def _tc_node_finish(x, hn, seg2, deg2, bm, wc, bc, o0, ob0, og, obb,
                    o1, ob1, fg, fb, wtl, wtr, wcbl, wcbr):
    return pl.pallas_call(
        _tc_node_finish_kernel,
        grid=(N_NODES // BN,),
        in_specs=[_rspec(BN, NODE_DIM), _rspec(BN, NODE_DIM),
                  _r3spec(BN, NODE_DIM, 0), _r3spec(BN, NODE_DIM, 1),
                  _r3spec(BN, 16, 0), _r3spec(BN, 16, 1),
                  _wspec(1, NODE_DIM),
                  _wspec(NODE_DIM, NODE_DIM), _wspec(1, NODE_DIM),
                  _wspec(NODE_DIM, NODE_DIM), _wspec(1, NODE_DIM),
                  _wspec(1, NODE_DIM), _wspec(1, NODE_DIM),
                  _wspec(NODE_DIM, NODE_DIM), _wspec(1, NODE_DIM),
                  _wspec(1, NODE_DIM), _wspec(1, NODE_DIM),
                  _wspec(NODE_DIM, INTER), _wspec(NODE_DIM, INTER),
                  _wspec(NODE_DIM, 16), _wspec(NODE_DIM, 16)],
        out_specs=[_rspec(BN, NODE_DIM), _rspec(BN, INTER), _rspec(BN, INTER),
                   _rspec(BN, 16), _rspec(BN, 16)],
        out_shape=(_f32(N_NODES, NODE_DIM), _f32(N_NODES, INTER),
                   _f32(N_NODES, INTER), _f32(N_NODES, 16),
                   _f32(N_NODES, 16)),
    )(x, hn, seg2, seg2, deg2, deg2, bm, wc, bc, o0, ob0, og, obb,
      o1, ob1, fg, fb, wtl, wtr, wcbl, wcbr)


P8 = 8
R8 = N_EDGES // P8
BEP = 2000


def _lnp(x, M, g, b):
    m = _dot(x, M)
    xc = x - m
    v = _dot(xc * xc, M)
    return xc * lax.rsqrt(v + 1e-5) * g + b


def _tc_edge_inter_kernel(hb, gl, gr, m32, bl, l0, lb0, lg, lbb, l1, lb1,
                          br, r0, rb0, rg, rbb, r1, rb1, mbl, mbr):
    hbb = hb[...]
    M = m32[...]
    il = _dot(hbb, bl[...]) + gl[...]
    h = _dot(il, l0[...]) + lb0[...]
    h = jnp.maximum(_lnp(h, M, lg[...], lbb[...]), 0.0)
    mbl[...] = _dot(h, l1[...]) + lb1[...]
    ir = _dot(hbb, br[...]) + gr[...]
    h = _dot(ir, r0[...]) + rb0[...]
    h = jnp.maximum(_lnp(h, M, rg[...], rbb[...]), 0.0)
    mbr[...] = _dot(h, r1[...]) + rb1[...]


def _tc_edge_inter(hb, gl, gr, *ws):
    return pl.pallas_call(
        _tc_edge_inter_kernel,
        grid=(R8 // BEP,),
        in_specs=[_rspec(BEP, 128), _rspec(BEP, 256), _rspec(BEP, 256),
                  _wspec(256, 256),
                  _wspec(128, 256), _wspec(256, 256),
                  _wspec(1, 256), _wspec(1, 256), _wspec(1, 256),
                  _wspec(256, 128), _wspec(1, 128),
                  _wspec(128, 256), _wspec(256, 256),
                  _wspec(1, 256), _wspec(1, 256), _wspec(1, 256),
                  _wspec(256, 128), _wspec(1, 128)],
        out_specs=[_rspec(BEP, 128), _rspec(BEP, 128)],
        out_shape=(_f32(R8, 128), _f32(R8, 128)),
    )(hb, gl, gr, *ws)


def _tc_edge_finish_kernel(gL, gR, hb, m16, ws, ball, o0, ob0, og, obb,
                           o1, ob1, fg, fb, out):
    hbb = hb[...]
    M = m16[...]
    upd = gL[...] + gR[...] + _dot(hbb, ws[...]) + ball[...]
    h = _dot(upd, o0[...]) + ob0[...]
    h = jnp.maximum(_lnp(h, M, og[...], obb[...]), 0.0)
    t = _dot(h, o1[...]) + ob1[...]
    out[...] = _lnp(t + hbb, M, fg[...], fb[...])


def _tc_edge_finish(gL, gR, hb, *ws):
    return pl.pallas_call(
        _tc_edge_finish_kernel,
        grid=(R8 // BEP,),
        in_specs=[_rspec(BEP, 128), _rspec(BEP, 128), _rspec(BEP, 128),
                  _wspec(128, 128),
                  _wspec(128, 128), _wspec(1, 128),
                  _wspec(128, 128), _wspec(1, 128),
                  _wspec(1, 128), _wspec(1, 128),
                  _wspec(128, 128), _wspec(1, 128),
                  _wspec(1, 128), _wspec(1, 128)],
        out_specs=_rspec(BEP, 128),
        out_shape=_f32(R8, 128),
    )(gL, gR, hb, *ws)


def _r1(v):
    return v.reshape(1, -1)


def _mlp_ws(p):
    return (p["l0"]["w"], _r1(p["l0"]["b"]), _r1(p["ln"]["g"]),
            _r1(p["ln"]["b"]), p["l1"]["w"], _r1(p["l1"]["b"]))


def _mlp_ws_fold(p, wm):
    return (p["l0"]["w"], _r1(p["l0"]["b"]), _r1(p["ln"]["g"]),
            _r1(p["ln"]["b"]), p["l1"]["w"] @ wm, _r1(p["l1"]["b"] @ wm))


def _bd(w):
    return jnp.kron(jnp.eye(P8, dtype=w.dtype), w)


def _t8(v):
    return jnp.tile(v.reshape(1, -1), (1, P8))


def _mlp_ws_p(p):
    return (_bd(p["l0"]["w"]), _t8(p["l0"]["b"]), _t8(p["ln"]["g"]),
            _t8(p["ln"]["b"]), _bd(p["l1"]["w"]), _t8(p["l1"]["b"]))


def _mlp_ws_fold_p(p, wm):
    return (_bd(p["l0"]["w"]), _t8(p["l0"]["b"]), _t8(p["ln"]["g"]),
            _t8(p["ln"]["b"]), _bd(p["l1"]["w"] @ wm), _t8(p["l1"]["b"] @ wm))


def kernel(h_node, h_edge, edge_index, node_extra, edge_extra, params):
    row3 = edge_index[0].reshape(NW, NB, BATCH)
    col3 = edge_index[1].reshape(NW, NB, BATCH)
    row3a = edge_index[0].reshape(NW, NB_A, BATCH_A)
    col3a = edge_index[1].reshape(NW, NB_A, BATCH_A)
    row16 = edge_index[0].reshape(NS, NBS, BATCH)
    col16 = edge_index[1].reshape(NS, NBS, BATCH)

    deg2 = _sc_deg(row3).reshape(2, N_PAD, 16)

    x, hb = h_node, h_edge
    for blk in params["blocks"]:
        npar, epar = blk["node"], blk["edge"]

        wm = npar["msg_net"]["w"]
        hn = _tc_node_mlp(x, *_mlp_ws_fold(npar["node_net"], wm))
        he = _tc_edge_mlp(hb, *_mlp_ws_fold(npar["edge_net"], wm))
        seg2 = _sc_node_agg(he.reshape(NW * NB_A, BATCH_A, NODE_DIM), hn,
                            row3a, col3a).reshape(2, N_PAD, NODE_DIM)
        x, tl, tr, cbl, cbr = _tc_node_finish(
            x, hn, seg2, deg2,
            _r1(npar["msg_net"]["b"]),
            npar["centroid_lin"]["w"], _r1(npar["centroid_lin"]["b"]),
            *_mlp_ws(npar["out_layer"]),
            _r1(npar["layer_norm"]["g"]), _r1(npar["layer_norm"]["b"]),
            epar["bond_ffn_left"]["node_linear"]["w"],
            epar["bond_ffn_right"]["node_linear"]["w"],
            epar["node_ffn_left"]["w"], epar["node_ffn_right"]["w"])

        gl3, gr3 = _sc_gather32(tl, tr, row3, col3)
        m32 = _bd(jnp.full((INTER, INTER), 1.0 / INTER, jnp.float32))
        mbl8, mbr8 = _tc_edge_inter(
            hb.reshape(R8, P8 * EDGE_DIM),
            gl3.reshape(R8, P8 * INTER), gr3.reshape(R8, P8 * INTER),
            m32,
            _bd(epar["bond_ffn_left"]["bond_linear"]["w"]),
            *_mlp_ws_fold_p(epar["bond_ffn_left"]["inter_module"],
                            epar["msg_left"]["w"]),
            _bd(epar["bond_ffn_right"]["bond_linear"]["w"]),
            *_mlp_ws_fold_p(epar["bond_ffn_right"]["inter_module"],
                            epar["msg_right"]["w"]))

        gL3, gR3 = _sc_scatter_gather(
            mbl8.reshape(NS * NBS, BATCH, EDGE_DIM),
            mbr8.reshape(NS * NBS, BATCH, EDGE_DIM),
            cbl, cbr, col16, row16, row16, col16)
        m16 = _bd(jnp.full((EDGE_DIM, EDGE_DIM), 1.0 / EDGE_DIM, jnp.float32))
        ball = _t8(epar["msg_left"]["b"] + epar["node_ffn_left"]["b"]
                   + epar["msg_right"]["b"] + epar["node_ffn_right"]["b"]
                   + epar["self_ffn"]["b"])
        hb = _tc_edge_finish(
            gL3.reshape(R8, P8 * EDGE_DIM), gR3.reshape(R8, P8 * EDGE_DIM),
            hb.reshape(R8, P8 * EDGE_DIM), m16,
            _bd(epar["self_ffn"]["w"]), ball,
            *_mlp_ws_p(epar["out_layer"]),
            _t8(epar["layer_norm"]["g"]),
            _t8(epar["layer_norm"]["b"])).reshape(N_EDGES, EDGE_DIM)

    return x, hb


# speedup vs baseline: 5.8292x; 1.0001x over previous
import functools

import jax
import jax.numpy as jnp
from jax import lax
from jax.experimental import pallas as pl
from jax.experimental.pallas import tpu as pltpu
from jax.experimental.pallas import tpu_sc as plsc

N_NODES = 10000
N_EDGES = 160000
NODE_DIM = 128
EDGE_DIM = 16
INTER = 32

NC, NS = 2, 16
NW = NC * NS
BATCH = 125
EPW = N_EDGES // NW
NB = EPW // BATCH
BATCH_A = 50
NB_A = EPW // BATCH_A
N_PAD = 10240
ROWS_PT = N_PAD // NS
ZCH = 40
EPS = N_EDGES // NS
NBS = EPS // BATCH
ROWS_LAST = N_NODES - (NS - 1) * ROWS_PT

_MESH = plsc.VectorSubcoreMesh(core_axis_name="c", subcore_axis_name="s")


def _f32(*shape):
    return jax.ShapeDtypeStruct(shape, jnp.float32)


def _bf16(*shape):
    return jax.ShapeDtypeStruct(shape, jnp.bfloat16)


def _fill(ref, rows, width, val):
    def body(r, _):
        for cidx in range(width // 16):
            ref[r, pl.ds(cidx * 16, 16)] = jnp.full((16,), val, jnp.float32)
        return 0
    lax.fori_loop(0, rows, body, 0)


def _worker_id():
    return lax.axis_index("s") * NC + lax.axis_index("c")


def _make_sc_deg():
    def body(row_i, out, iR, buf, zb, acc, sem):
        c = lax.axis_index("c")
        s = lax.axis_index("s")
        w = s * NC + c
        _fill(zb, ZCH, 16, 0.0)
        for z in range(ROWS_PT // ZCH):
            pltpu.sync_copy(zb, acc.at[pl.ds(s * ROWS_PT + z * ZCH, ZCH)])
        plsc.subcore_barrier()
        _fill(buf, BATCH, 16, 1.0)
        pltpu.sync_copy(row_i.at[w], iR)

        def step(j, _):
            pltpu.sync_copy(buf, acc.at[iR.at[j]], add=True)
            return 0
        lax.fori_loop(0, NB, step, 0)
        plsc.subcore_barrier()
        pltpu.sync_copy(acc.at[pl.ds(s * ROWS_PT, ROWS_PT)],
                        out.at[pl.ds(c * N_PAD + s * ROWS_PT, ROWS_PT)])

    return pl.kernel(
        body,
        compiler_params=pltpu.CompilerParams(use_tc_tiling_on_sc=False),
        out_type=_f32(2 * N_PAD, 16),
        mesh=_MESH,
        scratch_types=[
            pltpu.VMEM((NB, BATCH), jnp.int32),
            pltpu.VMEM((BATCH, 16), jnp.float32),
            pltpu.VMEM((ZCH, 16), jnp.float32),
            pltpu.VMEM_SHARED((N_PAD, 16), jnp.float32),
            pltpu.SemaphoreType.DMA,
        ],
    )


def _make_sc_node_agg():
    def body(he3, hn, row_i, col_i, out, rV, cV, bufA0, bufB0, bufA1, bufB1,
             zb, acc, semA0, semB0, semA1, semB1):
        c = lax.axis_index("c")
        s = lax.axis_index("s")
        w = s * NC + c
        _fill(zb, ZCH, NODE_DIM, 0.0)
        for z in range(ROWS_PT // ZCH):
            pltpu.sync_copy(zb, acc.at[pl.ds(s * ROWS_PT + z * ZCH, ZCH)])
        plsc.subcore_barrier()
        pltpu.sync_copy(row_i.at[w], rV)
        pltpu.sync_copy(col_i.at[w], cV)

        def issue(j, bufA, bufB, semA, semB):
            dA = pltpu.async_copy(he3.at[w * NB_A + j], bufA, semA)
            dB = pltpu.async_copy(hn.at[cV.at[j]], bufB, semB)
            return dA, dB

        def drain(j, dA, dB, bufA, bufB):
            dA.wait()
            dB.wait()
            pltpu.sync_copy(bufA, acc.at[rV.at[j]], add=True)
            pltpu.sync_copy(bufB, acc.at[rV.at[j]], add=True)

        d0 = issue(0, bufA0, bufB0, semA0, semB0)

        def pair(i, _):
            j0 = 2 * i
            d1 = issue(j0 + 1, bufA1, bufB1, semA1, semB1)
            drain(j0, *d0, bufA0, bufB0)
            issue(j0 + 2, bufA0, bufB0, semA0, semB0)
            drain(j0 + 1, *d1, bufA1, bufB1)
            return 0
        lax.fori_loop(0, NB_A // 2 - 1, pair, 0)
        d1 = issue(NB_A - 1, bufA1, bufB1, semA1, semB1)
        drain(NB_A - 2, *d0, bufA0, bufB0)
        drain(NB_A - 1, *d1, bufA1, bufB1)
        plsc.subcore_barrier()
        pltpu.sync_copy(acc.at[pl.ds(s * ROWS_PT, ROWS_PT)],
                        out.at[pl.ds(c * N_PAD + s * ROWS_PT, ROWS_PT)])

    return pl.kernel(
        body,
        compiler_params=pltpu.CompilerParams(use_tc_tiling_on_sc=False),
        out_type=_f32(2 * N_PAD, NODE_DIM),
        mesh=_MESH,
        scratch_types=[
            pltpu.VMEM((NB_A, BATCH_A), jnp.int32),
            pltpu.VMEM((NB_A, BATCH_A), jnp.int32),
            pltpu.VMEM((BATCH_A, NODE_DIM), jnp.float32),
            pltpu.VMEM((BATCH_A, NODE_DIM), jnp.float32),
            pltpu.VMEM((BATCH_A, NODE_DIM), jnp.float32),
            pltpu.VMEM((BATCH_A, NODE_DIM), jnp.float32),
            pltpu.VMEM((ZCH, NODE_DIM), jnp.float32),
            pltpu.VMEM_SHARED((N_PAD, NODE_DIM), jnp.float32),
            pltpu.SemaphoreType.DMA,
            pltpu.SemaphoreType.DMA,
            pltpu.SemaphoreType.DMA,
            pltpu.SemaphoreType.DMA,
        ],
    )


def _make_sc_gather2(wa, wb):
    def body(tabA, tabB, idxA, idxB, gA, gB, iA, iB, bufA0, bufB0,
             bufA1, bufB1, semA0, semB0, semA1, semB1):
        c = lax.axis_index("c")
        s = lax.axis_index("s")
        w = s * NC + c
        pltpu.sync_copy(idxA.at[w], iA)
        pltpu.sync_copy(idxB.at[w], iB)

        def issue(j, bufA, bufB, semA, semB):
            dA = pltpu.async_copy(tabA.at[iA.at[j]], bufA, semA)
            dB = pltpu.async_copy(tabB.at[iB.at[j]], bufB, semB)
            return dA, dB

        def drain(j, dA, dB, bufA, bufB):
            dA.wait()
            pltpu.sync_copy(bufA, gA.at[w * NB + j])
            dB.wait()
            pltpu.sync_copy(bufB, gB.at[w * NB + j])

        d0 = issue(0, bufA0, bufB0, semA0, semB0)

        def pair(i, _):
            j0 = 2 * i
            d1 = issue(j0 + 1, bufA1, bufB1, semA1, semB1)
            drain(j0, *d0, bufA0, bufB0)
            issue(j0 + 2, bufA0, bufB0, semA0, semB0)
            drain(j0 + 1, *d1, bufA1, bufB1)
            return 0
        lax.fori_loop(0, NB // 2 - 1, pair, 0)
        d1 = issue(NB - 1, bufA1, bufB1, semA1, semB1)
        drain(NB - 2, *d0, bufA0, bufB0)
        drain(NB - 1, *d1, bufA1, bufB1)

    return pl.kernel(
        body,
        compiler_params=pltpu.CompilerParams(use_tc_tiling_on_sc=False),
        out_type=(_f32(NW * NB, BATCH, wa), _f32(NW * NB, BATCH, wb)),
        mesh=_MESH,
        scratch_types=[
            pltpu.VMEM((NB, BATCH), jnp.int32),
            pltpu.VMEM((NB, BATCH), jnp.int32),
            pltpu.VMEM((BATCH, wa), jnp.float32),
            pltpu.VMEM((BATCH, wb), jnp.float32),
            pltpu.VMEM((BATCH, wa), jnp.float32),
            pltpu.VMEM((BATCH, wb), jnp.float32),
            pltpu.SemaphoreType.DMA,
            pltpu.SemaphoreType.DMA,
            pltpu.SemaphoreType.DMA,
            pltpu.SemaphoreType.DMA,
        ],
    )


def _make_sc_scatter_gather():
    def body(vA3, vB3, cbl, cbr, isA, isB, igA, igB, gA, gB,
             iS, iG, buf0, buf1, acc, sem0, sem1):
        c = lax.axis_index("c")
        s = lax.axis_index("s")

        def run(v3, cb, iS_h, iG_h, gout):
            pltpu.sync_copy(iS_h.at[s], iS)
            pltpu.sync_copy(iG_h.at[s], iG)

            @pl.when(s < NS - 1)
            def _():
                pltpu.sync_copy(cb.at[pl.ds(s * ROWS_PT, ROWS_PT)],
                                acc.at[pl.ds(s * ROWS_PT, ROWS_PT)])

            @pl.when(s == NS - 1)
            def _():
                pltpu.sync_copy(
                    cb.at[pl.ds((NS - 1) * ROWS_PT, ROWS_LAST)],
                    acc.at[pl.ds((NS - 1) * ROWS_PT, ROWS_LAST)])
            plsc.subcore_barrier()

            def issue(j, buf, sem):
                return pltpu.async_copy(v3.at[s * NBS + j], buf, sem)

            def drain(j, d, buf):
                d.wait()
                pltpu.sync_copy(buf, acc.at[iS.at[j]], add=True)

            d0 = issue(0, buf0, sem0)

            def pair(i, _):
                j0 = 2 * i
                d1 = issue(j0 + 1, buf1, sem1)
                drain(j0, d0, buf0)
                issue(j0 + 2, buf0, sem0)
                drain(j0 + 1, d1, buf1)
                return 0
            lax.fori_loop(0, NBS // 2 - 1, pair, 0)
            d1 = issue(NBS - 1, buf1, sem1)
            drain(NBS - 2, d0, buf0)
            drain(NBS - 1, d1, buf1)
            plsc.subcore_barrier()

            def gissue(j, buf, sem):
                return pltpu.async_copy(acc.at[iG.at[j]], buf, sem)

            def gdrain(j, d, buf):
                d.wait()
                pltpu.sync_copy(buf, gout.at[s * NBS + j])

            g0 = gissue(0, buf0, sem0)

            def gpair(i, _):
                j0 = 2 * i
                g1 = gissue(j0 + 1, buf1, sem1)
                gdrain(j0, g0, buf0)
                gissue(j0 + 2, buf0, sem0)
                gdrain(j0 + 1, g1, buf1)
                return 0
            lax.fori_loop(0, NBS // 2 - 1, gpair, 0)
            g1 = gissue(NBS - 1, buf1, sem1)
            gdrain(NBS - 2, g0, buf0)
            gdrain(NBS - 1, g1, buf1)

        @pl.when(c == 0)
        def _():
            run(vA3, cbl, isA, igA, gA)

        @pl.when(c == 1)
        def _():
            run(vB3, cbr, isB, igB, gB)

    return pl.kernel(
        body,
        compiler_params=pltpu.CompilerParams(use_tc_tiling_on_sc=False),
        out_type=(_f32(NS * NBS, BATCH, EDGE_DIM),
                  _f32(NS * NBS, BATCH, EDGE_DIM)),
        mesh=_MESH,
        scratch_types=[
            pltpu.VMEM((NBS, BATCH), jnp.int32),
            pltpu.VMEM((NBS, BATCH), jnp.int32),
            pltpu.VMEM((BATCH, EDGE_DIM), jnp.float32),
            pltpu.VMEM((BATCH, EDGE_DIM), jnp.float32),
            pltpu.VMEM_SHARED((N_PAD, EDGE_DIM), jnp.float32),
            pltpu.SemaphoreType.DMA,
            pltpu.SemaphoreType.DMA,
        ],
    )


_sc_deg = _make_sc_deg()
_sc_node_agg = _make_sc_node_agg()
_sc_gather32 = _make_sc_gather2(INTER, INTER)
_sc_scatter_gather = _make_sc_scatter_gather()


def _ln(x, g, b):
    m = jnp.mean(x, axis=-1, keepdims=True)
    v = jnp.mean((x - m) ** 2, axis=-1, keepdims=True)
    return (x - m) * lax.rsqrt(v + 1e-5) * g + b


def _dot(x, w):
    return jnp.dot(x, w, preferred_element_type=jnp.float32)


BN = 5000
BE = 8000


def _wspec(r, c):
    return pl.BlockSpec((r, c), lambda i: (0, 0))


def _rspec(rows, cols):
    return pl.BlockSpec((rows, cols), lambda i: (i, 0))


def _r3spec(rows, cols, half):
    return pl.BlockSpec((1, rows, cols), lambda i, h=half: (h, i, 0))


def _tc_mlp_kernel(x, w0, b0, g, bb, w1, b1, o):
    h = _dot(x[...], w0[...]) + b0[...]
    h = jnp.maximum(_ln(h, g[...], bb[...]), 0.0)
    o[...] = _dot(h, w1[...]) + b1[...]


def _tc_node_mlp(x, w0, b0, g, bb, w1, b1):
    return pl.pallas_call(
        _tc_mlp_kernel,
        grid=(N_NODES // BN,),
        in_specs=[_rspec(BN, NODE_DIM), _wspec(NODE_DIM, NODE_DIM),
                  _wspec(1, NODE_DIM), _wspec(1, NODE_DIM),
                  _wspec(1, NODE_DIM), _wspec(NODE_DIM, NODE_DIM),
                  _wspec(1, NODE_DIM)],
        out_specs=_rspec(BN, NODE_DIM),
        out_shape=_f32(N_NODES, NODE_DIM),
    )(x, w0, b0, g, bb, w1, b1)


def _tc_edge_mlp(x, w0, b0, g, bb, w1, b1):
    return pl.pallas_call(
        _tc_mlp_kernel,
        grid=(N_EDGES // BE,),
        in_specs=[_rspec(BE, EDGE_DIM), _wspec(EDGE_DIM, NODE_DIM),
                  _wspec(1, NODE_DIM), _wspec(1, NODE_DIM),
                  _wspec(1, NODE_DIM), _wspec(NODE_DIM, NODE_DIM),
                  _wspec(1, NODE_DIM)],
        out_specs=_rspec(BE, NODE_DIM),
        out_shape=_f32(N_EDGES, NODE_DIM),
    )(x, w0, b0, g, bb, w1, b1)


def _tc_node_finish_kernel(x, hn, seg_lo, seg_hi, deg_lo, deg_hi,
                           bm, wc, bc, o0, ob0, og, obb, o1, ob1, fg, fb,
                           wtl, wtr, wcbl, wcbr,
                           xn, tl, tr, cbl, cbr):
    seg = seg_lo[0] + seg_hi[0]
    deg = deg_lo[0][:, :1] + deg_hi[0][:, :1]
    xb = x[...]
    aggr = seg + deg * hn[...] + deg * bm[...]
    t = _dot(xb, wc[...]) + bc[...] + aggr
    h = _dot(t, o0[...]) + ob0[...]
    h = jnp.maximum(_ln(h, og[...], obb[...]), 0.0)
    t2 = _dot(h, o1[...]) + ob1[...]
    y = _ln(t2 + xb, fg[...], fb[...])
    xn[...] = y
    tl[...] = _dot(y, wtl[...])
    tr[...] = _dot(y, wtr[...])
    cbl[...] = _dot(y, wcbl[...])
    cbr[...] = _dot(y, wcbr[...])


def _tc_node_finish(x, hn, seg2, deg2, bm, wc, bc, o0, ob0, og, obb,
                    o1, ob1, fg, fb, wtl, wtr, wcbl, wcbr):
    return pl.pallas_call(
        _tc_node_finish_kernel,
        grid=(N_NODES // BN,),
        in_specs=[_rspec(BN, NODE_DIM), _rspec(BN, NODE_DIM),
                  _r3spec(BN, NODE_DIM, 0), _r3spec(BN, NODE_DIM, 1),
                  _r3spec(BN, 16, 0), _r3spec(BN, 16, 1),
                  _wspec(1, NODE_DIM),
                  _wspec(NODE_DIM, NODE_DIM), _wspec(1, NODE_DIM),
                  _wspec(NODE_DIM, NODE_DIM), _wspec(1, NODE_DIM),
                  _wspec(1, NODE_DIM), _wspec(1, NODE_DIM),
                  _wspec(NODE_DIM, NODE_DIM), _wspec(1, NODE_DIM),
                  _wspec(1, NODE_DIM), _wspec(1, NODE_DIM),
                  _wspec(NODE_DIM, INTER), _wspec(NODE_DIM, INTER),
                  _wspec(NODE_DIM, 16), _wspec(NODE_DIM, 16)],
        out_specs=[_rspec(BN, NODE_DIM), _rspec(BN, INTER), _rspec(BN, INTER),
                   _rspec(BN, 16), _rspec(BN, 16)],
        out_shape=(_f32(N_NODES, NODE_DIM), _f32(N_NODES, INTER),
                   _f32(N_NODES, INTER), _f32(N_NODES, 16),
                   _f32(N_NODES, 16)),
    )(x, hn, seg2, seg2, deg2, deg2, bm, wc, bc, o0, ob0, og, obb,
      o1, ob1, fg, fb, wtl, wtr, wcbl, wcbr)


P8 = 8
R8 = N_EDGES // P8
BEP = 2000


def _lnp(x, M, g, b):
    m = _dot(x, M)
    xc = x - m
    v = _dot(xc * xc, M)
    return xc * lax.rsqrt(v + 1e-5) * g + b


def _tc_edge_inter_kernel(hb, gl, gr, m32, bl, l0, lb0, lg, lbb, l1, lb1,
                          br, r0, rb0, rg, rbb, r1, rb1, mbl, mbr):
    hbb = hb[...]
    M = m32[...]
    il = _dot(hbb, bl[...]) + gl[...]
    h = _dot(il, l0[...]) + lb0[...]
    h = jnp.maximum(_lnp(h, M, lg[...], lbb[...]), 0.0)
    mbl[...] = _dot(h, l1[...]) + lb1[...]
    ir = _dot(hbb, br[...]) + gr[...]
    h = _dot(ir, r0[...]) + rb0[...]
    h = jnp.maximum(_lnp(h, M, rg[...], rbb[...]), 0.0)
    mbr[...] = _dot(h, r1[...]) + rb1[...]


def _tc_edge_inter(hb, gl, gr, *ws):
    return pl.pallas_call(
        _tc_edge_inter_kernel,
        grid=(R8 // BEP,),
        in_specs=[_rspec(BEP, 128), _rspec(BEP, 256), _rspec(BEP, 256),
                  _wspec(256, 256),
                  _wspec(128, 256), _wspec(256, 256),
                  _wspec(1, 256), _wspec(1, 256), _wspec(1, 256),
                  _wspec(256, 128), _wspec(1, 128),
                  _wspec(128, 256), _wspec(256, 256),
                  _wspec(1, 256), _wspec(1, 256), _wspec(1, 256),
                  _wspec(256, 128), _wspec(1, 128)],
        out_specs=[_rspec(BEP, 128), _rspec(BEP, 128)],
        out_shape=(_f32(R8, 128), _f32(R8, 128)),
    )(hb, gl, gr, *ws)


def _tc_edge_finish_kernel(gL, gR, hb, m16, ws, ball, o0, ob0, og, obb,
                           o1, ob1, fg, fb, out):
    hbb = hb[...]
    M = m16[...]
    upd = gL[...] + gR[...] + _dot(hbb, ws[...]) + ball[...]
    h = _dot(upd, o0[...]) + ob0[...]
    h = jnp.maximum(_lnp(h, M, og[...], obb[...]), 0.0)
    t = _dot(h, o1[...]) + ob1[...]
    out[...] = _lnp(t + hbb, M, fg[...], fb[...])


def _tc_edge_finish(gL, gR, hb, *ws):
    return pl.pallas_call(
        _tc_edge_finish_kernel,
        grid=(R8 // BEP,),
        in_specs=[_rspec(BEP, 128), _rspec(BEP, 128), _rspec(BEP, 128),
                  _wspec(128, 128),
                  _wspec(128, 128), _wspec(1, 128),
                  _wspec(128, 128), _wspec(1, 128),
                  _wspec(1, 128), _wspec(1, 128),
                  _wspec(128, 128), _wspec(1, 128),
                  _wspec(1, 128), _wspec(1, 128)],
        out_specs=_rspec(BEP, 128),
        out_shape=_f32(R8, 128),
    )(gL, gR, hb, *ws)


def _r1(v):
    return v.reshape(1, -1)


def _mlp_ws(p):
    return (p["l0"]["w"], _r1(p["l0"]["b"]), _r1(p["ln"]["g"]),
            _r1(p["ln"]["b"]), p["l1"]["w"], _r1(p["l1"]["b"]))


def _mlp_ws_fold(p, wm):
    return (p["l0"]["w"], _r1(p["l0"]["b"]), _r1(p["ln"]["g"]),
            _r1(p["ln"]["b"]), p["l1"]["w"] @ wm, _r1(p["l1"]["b"] @ wm))


def _bd(w):
    return jnp.kron(jnp.eye(P8, dtype=w.dtype), w)


def _t8(v):
    return jnp.tile(v.reshape(1, -1), (1, P8))


def _mlp_ws_p(p):
    return (_bd(p["l0"]["w"]), _t8(p["l0"]["b"]), _t8(p["ln"]["g"]),
            _t8(p["ln"]["b"]), _bd(p["l1"]["w"]), _t8(p["l1"]["b"]))


def _mlp_ws_fold_p(p, wm):
    return (_bd(p["l0"]["w"]), _t8(p["l0"]["b"]), _t8(p["ln"]["g"]),
            _t8(p["ln"]["b"]), _bd(p["l1"]["w"] @ wm), _t8(p["l1"]["b"] @ wm))


def kernel(h_node, h_edge, edge_index, node_extra, edge_extra, params):
    row3 = edge_index[0].reshape(NW, NB, BATCH)
    col3 = edge_index[1].reshape(NW, NB, BATCH)
    row3a = edge_index[0].reshape(NW, NB_A, BATCH_A)
    col3a = edge_index[1].reshape(NW, NB_A, BATCH_A)
    row16 = edge_index[0].reshape(NS, NBS, BATCH)
    col16 = edge_index[1].reshape(NS, NBS, BATCH)

    deg2 = _sc_deg(row3).reshape(2, N_PAD, 16)

    x, hb = h_node, h_edge
    for blk in params["blocks"]:
        npar, epar = blk["node"], blk["edge"]

        wm = npar["msg_net"]["w"]
        hn = _tc_node_mlp(x, *_mlp_ws_fold(npar["node_net"], wm))
        he = _tc_edge_mlp(hb, *_mlp_ws_fold(npar["edge_net"], wm))
        seg2 = _sc_node_agg(he.reshape(NW * NB_A, BATCH_A, NODE_DIM), hn,
                            row3a, col3a).reshape(2, N_PAD, NODE_DIM)
        x, tl, tr, cbl, cbr = _tc_node_finish(
            x, hn, seg2, deg2,
            _r1(npar["msg_net"]["b"]),
            npar["centroid_lin"]["w"], _r1(npar["centroid_lin"]["b"]),
            *_mlp_ws(npar["out_layer"]),
            _r1(npar["layer_norm"]["g"]), _r1(npar["layer_norm"]["b"]),
            epar["bond_ffn_left"]["node_linear"]["w"],
            epar["bond_ffn_right"]["node_linear"]["w"],
            epar["node_ffn_left"]["w"], epar["node_ffn_right"]["w"])

        gl3, gr3 = _sc_gather32(tl, tr, row3, col3)
        m32 = _bd(jnp.full((INTER, INTER), 1.0 / INTER, jnp.float32))
        mbl8, mbr8 = _tc_edge_inter(
            hb.reshape(R8, P8 * EDGE_DIM),
            gl3.reshape(R8, P8 * INTER), gr3.reshape(R8, P8 * INTER),
            m32,
            _bd(epar["bond_ffn_left"]["bond_linear"]["w"]),
            *_mlp_ws_fold_p(epar["bond_ffn_left"]["inter_module"],
                            epar["msg_left"]["w"]),
            _bd(epar["bond_ffn_right"]["bond_linear"]["w"]),
            *_mlp_ws_fold_p(epar["bond_ffn_right"]["inter_module"],
                            epar["msg_right"]["w"]))

        gL3, gR3 = _sc_scatter_gather(
            mbl8.reshape(NS * NBS, BATCH, EDGE_DIM),
            mbr8.reshape(NS * NBS, BATCH, EDGE_DIM),
            cbl, cbr, col16, row16, row16, col16)
        m16 = _bd(jnp.full((EDGE_DIM, EDGE_DIM), 1.0 / EDGE_DIM, jnp.float32))
        ball = _t8(epar["msg_left"]["b"] + epar["node_ffn_left"]["b"]
                   + epar["msg_right"]["b"] + epar["node_ffn_right"]["b"]
                   + epar["self_ffn"]["b"])
        hb = _tc_edge_finish(
            gL3.reshape(R8, P8 * EDGE_DIM), gR3.reshape(R8, P8 * EDGE_DIM),
            hb.reshape(R8, P8 * EDGE_DIM), m16,
            _bd(epar["self_ffn"]["w"]), ball,
            *_mlp_ws_p(epar["out_layer"]),
            _t8(epar["layer_norm"]["g"]),
            _t8(epar["layer_norm"]["b"])).reshape(N_EDGES, EDGE_DIM)

    return x, hb


# speedup vs baseline: 6.2862x; 1.0784x over previous
import functools

import jax
import jax.numpy as jnp
from jax import lax
from jax.experimental import pallas as pl
from jax.experimental.pallas import tpu as pltpu
from jax.experimental.pallas import tpu_sc as plsc

N_NODES = 10000
N_EDGES = 160000
NODE_DIM = 128
EDGE_DIM = 16
INTER = 32

NC, NS = 2, 16
NW = NC * NS
BATCH = 125
EPW = N_EDGES // NW
NB = EPW // BATCH
N_PAD = 10240
ROWS_PT = N_PAD // NS
ZCH = 40
EPS = N_EDGES // NS
NBS = EPS // BATCH
ROWS_LAST = N_NODES - (NS - 1) * ROWS_PT

_MESH = plsc.VectorSubcoreMesh(core_axis_name="c", subcore_axis_name="s")


def _f32(*shape):
    return jax.ShapeDtypeStruct(shape, jnp.float32)


def _bf16(*shape):
    return jax.ShapeDtypeStruct(shape, jnp.bfloat16)


def _fill(ref, rows, width, val):
    def body(r, _):
        for cidx in range(width // 16):
            ref[r, pl.ds(cidx * 16, 16)] = jnp.full((16,), val, jnp.float32)
        return 0
    lax.fori_loop(0, rows, body, 0)


def _worker_id():
    return lax.axis_index("s") * NC + lax.axis_index("c")


def _make_sc_deg():
    def body(row_i, out, iR, buf, zb, acc, sem):
        c = lax.axis_index("c")
        s = lax.axis_index("s")
        w = s * NC + c
        _fill(zb, ZCH, 16, 0.0)
        for z in range(ROWS_PT // ZCH):
            pltpu.sync_copy(zb, acc.at[pl.ds(s * ROWS_PT + z * ZCH, ZCH)])
        plsc.subcore_barrier()
        _fill(buf, BATCH, 16, 1.0)
        pltpu.sync_copy(row_i.at[w], iR)

        def step(j, _):
            pltpu.sync_copy(buf, acc.at[iR.at[j]], add=True)
            return 0
        lax.fori_loop(0, NB, step, 0)
        plsc.subcore_barrier()
        pltpu.sync_copy(acc.at[pl.ds(s * ROWS_PT, ROWS_PT)],
                        out.at[pl.ds(c * N_PAD + s * ROWS_PT, ROWS_PT)])

    return pl.kernel(
        body,
        compiler_params=pltpu.CompilerParams(use_tc_tiling_on_sc=False),
        out_type=_f32(2 * N_PAD, 16),
        mesh=_MESH,
        scratch_types=[
            pltpu.VMEM((NB, BATCH), jnp.int32),
            pltpu.VMEM((BATCH, 16), jnp.float32),
            pltpu.VMEM((ZCH, 16), jnp.float32),
            pltpu.VMEM_SHARED((N_PAD, 16), jnp.float32),
            pltpu.SemaphoreType.DMA,
        ],
    )


def _make_sc_seg_agg(gather):
    def body(src, row_i, col_i, out, rV, cV, buf0, buf1, zb, acc, sem0, sem1):
        c = lax.axis_index("c")
        s = lax.axis_index("s")
        w = s * NC + c
        _fill(zb, ZCH, NODE_DIM, 0.0)
        for z in range(ROWS_PT // ZCH):
            pltpu.sync_copy(zb, acc.at[pl.ds(s * ROWS_PT + z * ZCH, ZCH)])
        plsc.subcore_barrier()
        pltpu.sync_copy(row_i.at[w], rV)
        if gather:
            pltpu.sync_copy(col_i.at[w], cV)

        def issue(j, buf, sem):
            if gather:
                return pltpu.async_copy(src.at[cV.at[j]], buf, sem)
            return pltpu.async_copy(src.at[w * NB + j], buf, sem)

        def drain(j, d, buf):
            d.wait()
            pltpu.sync_copy(buf, acc.at[rV.at[j]], add=True)

        d0 = issue(0, buf0, sem0)

        def pair(i, _):
            j0 = 2 * i
            d1 = issue(j0 + 1, buf1, sem1)
            drain(j0, d0, buf0)
            issue(j0 + 2, buf0, sem0)
            drain(j0 + 1, d1, buf1)
            return 0
        lax.fori_loop(0, NB // 2 - 1, pair, 0)
        d1 = issue(NB - 1, buf1, sem1)
        drain(NB - 2, d0, buf0)
        drain(NB - 1, d1, buf1)
        plsc.subcore_barrier()
        pltpu.sync_copy(acc.at[pl.ds(s * ROWS_PT, ROWS_PT)],
                        out.at[pl.ds(c * N_PAD + s * ROWS_PT, ROWS_PT)])

    return pl.kernel(
        body,
        compiler_params=pltpu.CompilerParams(use_tc_tiling_on_sc=False),
        out_type=_f32(2 * N_PAD, NODE_DIM),
        mesh=_MESH,
        scratch_types=[
            pltpu.VMEM((NB, BATCH), jnp.int32),
            pltpu.VMEM((NB, BATCH), jnp.int32),
            pltpu.VMEM((BATCH, NODE_DIM), jnp.float32),
            pltpu.VMEM((BATCH, NODE_DIM), jnp.float32),
            pltpu.VMEM((ZCH, NODE_DIM), jnp.float32),
            pltpu.VMEM_SHARED((N_PAD, NODE_DIM), jnp.float32),
            pltpu.SemaphoreType.DMA,
            pltpu.SemaphoreType.DMA,
        ],
    )


def _make_sc_gather2(wa, wb):
    def body(tabA, tabB, idxA, idxB, gA, gB, iA, iB, bufA0, bufB0,
             bufA1, bufB1, semA0, semB0, semA1, semB1):
        c = lax.axis_index("c")
        s = lax.axis_index("s")
        w = s * NC + c
        pltpu.sync_copy(idxA.at[w], iA)
        pltpu.sync_copy(idxB.at[w], iB)

        def issue(j, bufA, bufB, semA, semB):
            dA = pltpu.async_copy(tabA.at[iA.at[j]], bufA, semA)
            dB = pltpu.async_copy(tabB.at[iB.at[j]], bufB, semB)
            return dA, dB

        def drain(j, dA, dB, bufA, bufB):
            dA.wait()
            pltpu.sync_copy(bufA, gA.at[w * NB + j])
            dB.wait()
            pltpu.sync_copy(bufB, gB.at[w * NB + j])

        d0 = issue(0, bufA0, bufB0, semA0, semB0)

        def pair(i, _):
            j0 = 2 * i
            d1 = issue(j0 + 1, bufA1, bufB1, semA1, semB1)
            drain(j0, *d0, bufA0, bufB0)
            issue(j0 + 2, bufA0, bufB0, semA0, semB0)
            drain(j0 + 1, *d1, bufA1, bufB1)
            return 0
        lax.fori_loop(0, NB // 2 - 1, pair, 0)
        d1 = issue(NB - 1, bufA1, bufB1, semA1, semB1)
        drain(NB - 2, *d0, bufA0, bufB0)
        drain(NB - 1, *d1, bufA1, bufB1)

    return pl.kernel(
        body,
        compiler_params=pltpu.CompilerParams(use_tc_tiling_on_sc=False),
        out_type=(_f32(NW * NB, BATCH, wa), _f32(NW * NB, BATCH, wb)),
        mesh=_MESH,
        scratch_types=[
            pltpu.VMEM((NB, BATCH), jnp.int32),
            pltpu.VMEM((NB, BATCH), jnp.int32),
            pltpu.VMEM((BATCH, wa), jnp.float32),
            pltpu.VMEM((BATCH, wb), jnp.float32),
            pltpu.VMEM((BATCH, wa), jnp.float32),
            pltpu.VMEM((BATCH, wb), jnp.float32),
            pltpu.SemaphoreType.DMA,
            pltpu.SemaphoreType.DMA,
            pltpu.SemaphoreType.DMA,
            pltpu.SemaphoreType.DMA,
        ],
    )


def _make_sc_scatter_gather():
    def body(vA3, vB3, cbl, cbr, isA, isB, igA, igB, gA, gB,
             iS, iG, buf0, buf1, acc, sem0, sem1):
        c = lax.axis_index("c")
        s = lax.axis_index("s")

        def run(v3, cb, iS_h, iG_h, gout):
            pltpu.sync_copy(iS_h.at[s], iS)
            pltpu.sync_copy(iG_h.at[s], iG)

            @pl.when(s < NS - 1)
            def _():
                pltpu.sync_copy(cb.at[pl.ds(s * ROWS_PT, ROWS_PT)],
                                acc.at[pl.ds(s * ROWS_PT, ROWS_PT)])

            @pl.when(s == NS - 1)
            def _():
                pltpu.sync_copy(
                    cb.at[pl.ds((NS - 1) * ROWS_PT, ROWS_LAST)],
                    acc.at[pl.ds((NS - 1) * ROWS_PT, ROWS_LAST)])
            plsc.subcore_barrier()

            def issue(j, buf, sem):
                return pltpu.async_copy(v3.at[s * NBS + j], buf, sem)

            def drain(j, d, buf):
                d.wait()
                pltpu.sync_copy(buf, acc.at[iS.at[j]], add=True)

            d0 = issue(0, buf0, sem0)

            def pair(i, _):
                j0 = 2 * i
                d1 = issue(j0 + 1, buf1, sem1)
                drain(j0, d0, buf0)
                issue(j0 + 2, buf0, sem0)
                drain(j0 + 1, d1, buf1)
                return 0
            lax.fori_loop(0, NBS // 2 - 1, pair, 0)
            d1 = issue(NBS - 1, buf1, sem1)
            drain(NBS - 2, d0, buf0)
            drain(NBS - 1, d1, buf1)
            plsc.subcore_barrier()

            def gissue(j, buf, sem):
                return pltpu.async_copy(acc.at[iG.at[j]], buf, sem)

            def gdrain(j, d, buf):
                d.wait()
                pltpu.sync_copy(buf, gout.at[s * NBS + j])

            g0 = gissue(0, buf0, sem0)

            def gpair(i, _):
                j0 = 2 * i
                g1 = gissue(j0 + 1, buf1, sem1)
                gdrain(j0, g0, buf0)
                gissue(j0 + 2, buf0, sem0)
                gdrain(j0 + 1, g1, buf1)
                return 0
            lax.fori_loop(0, NBS // 2 - 1, gpair, 0)
            g1 = gissue(NBS - 1, buf1, sem1)
            gdrain(NBS - 2, g0, buf0)
            gdrain(NBS - 1, g1, buf1)

        @pl.when(c == 0)
        def _():
            run(vA3, cbl, isA, igA, gA)

        @pl.when(c == 1)
        def _():
            run(vB3, cbr, isB, igB, gB)

    return pl.kernel(
        body,
        compiler_params=pltpu.CompilerParams(use_tc_tiling_on_sc=False),
        out_type=(_f32(NS * NBS, BATCH, EDGE_DIM),
                  _f32(NS * NBS, BATCH, EDGE_DIM)),
        mesh=_MESH,
        scratch_types=[
            pltpu.VMEM((NBS, BATCH), jnp.int32),
            pltpu.VMEM((NBS, BATCH), jnp.int32),
            pltpu.VMEM((BATCH, EDGE_DIM), jnp.float32),
            pltpu.VMEM((BATCH, EDGE_DIM), jnp.float32),
            pltpu.VMEM_SHARED((N_PAD, EDGE_DIM), jnp.float32),
            pltpu.SemaphoreType.DMA,
            pltpu.SemaphoreType.DMA,
        ],
    )


_sc_deg = _make_sc_deg()
_sc_agg_gather = _make_sc_seg_agg(gather=True)
_sc_agg_stream = _make_sc_seg_agg(gather=False)
_sc_gather32 = _make_sc_gather2(INTER, INTER)
_sc_scatter_gather = _make_sc_scatter_gather()


def _ln(x, g, b):
    m = jnp.mean(x, axis=-1, keepdims=True)
    v = jnp.mean((x - m) ** 2, axis=-1, keepdims=True)
    return (x - m) * lax.rsqrt(v + 1e-5) * g + b


def _dot(x, w):
    return jnp.dot(x, w, preferred_element_type=jnp.float32)


BN = 2000
BE = 8000


def _wspec(r, c):
    return pl.BlockSpec((r, c), lambda i: (0, 0))


def _rspec(rows, cols):
    return pl.BlockSpec((rows, cols), lambda i: (i, 0))


def _r3spec(rows, cols, half):
    return pl.BlockSpec((1, rows, cols), lambda i, h=half: (h, i, 0))


def _tc_mlp_kernel(x, w0, b0, g, bb, w1, b1, o):
    h = _dot(x[...], w0[...]) + b0[...]
    h = jnp.maximum(_ln(h, g[...], bb[...]), 0.0)
    o[...] = _dot(h, w1[...]) + b1[...]


def _tc_node_mlp(x, w0, b0, g, bb, w1, b1):
    return pl.pallas_call(
        _tc_mlp_kernel,
        grid=(N_NODES // BN,),
        in_specs=[_rspec(BN, NODE_DIM), _wspec(NODE_DIM, NODE_DIM),
                  _wspec(1, NODE_DIM), _wspec(1, NODE_DIM),
                  _wspec(1, NODE_DIM), _wspec(NODE_DIM, NODE_DIM),
                  _wspec(1, NODE_DIM)],
        out_specs=_rspec(BN, NODE_DIM),
        out_shape=_f32(N_NODES, NODE_DIM),
    )(x, w0, b0, g, bb, w1, b1)


def _tc_edge_mlp(x, w0, b0, g, bb, w1, b1):
    return pl.pallas_call(
        _tc_mlp_kernel,
        grid=(N_EDGES // BE,),
        in_specs=[_rspec(BE, EDGE_DIM), _wspec(EDGE_DIM, NODE_DIM),
                  _wspec(1, NODE_DIM), _wspec(1, NODE_DIM),
                  _wspec(1, NODE_DIM), _wspec(NODE_DIM, NODE_DIM),
                  _wspec(1, NODE_DIM)],
        out_specs=_rspec(BE, NODE_DIM),
        out_shape=_f32(N_EDGES, NODE_DIM),
    )(x, w0, b0, g, bb, w1, b1)


def _tc_node_finish_kernel(x, hn, sa_lo, sa_hi, sb_lo, sb_hi, deg_lo, deg_hi,
                           bm, wc, bc, o0, ob0, og, obb, o1, ob1, fg, fb,
                           wtl, wtr, wcbl, wcbr,
                           xn, tl, tr, cbl, cbr):
    seg = sa_lo[0] + sa_hi[0] + sb_lo[0] + sb_hi[0]
    deg = deg_lo[0][:, :1] + deg_hi[0][:, :1]
    xb = x[...]
    aggr = seg + deg * hn[...] + deg * bm[...]
    t = _dot(xb, wc[...]) + bc[...] + aggr
    h = _dot(t, o0[...]) + ob0[...]
    h = jnp.maximum(_ln(h, og[...], obb[...]), 0.0)
    t2 = _dot(h, o1[...]) + ob1[...]
    y = _ln(t2 + xb, fg[...], fb[...])
    xn[...] = y
    tl[...] = _dot(y, wtl[...])
    tr[...] = _dot(y, wtr[...])
    cbl[...] = _dot(y, wcbl[...])
    cbr[...] = _dot(y, wcbr[...])


def _tc_node_finish(x, hn, sega2, segb2, deg2, bm, wc, bc, o0, ob0, og, obb,
                    o1, ob1, fg, fb, wtl, wtr, wcbl, wcbr):
    return pl.pallas_call(
        _tc_node_finish_kernel,
        grid=(N_NODES // BN,),
        in_specs=[_rspec(BN, NODE_DIM), _rspec(BN, NODE_DIM),
                  _r3spec(BN, NODE_DIM, 0), _r3spec(BN, NODE_DIM, 1),
                  _r3spec(BN, NODE_DIM, 0), _r3spec(BN, NODE_DIM, 1),
                  _r3spec(BN, 16, 0), _r3spec(BN, 16, 1),
                  _wspec(1, NODE_DIM),
                  _wspec(NODE_DIM, NODE_DIM), _wspec(1, NODE_DIM),
                  _wspec(NODE_DIM, NODE_DIM), _wspec(1, NODE_DIM),
                  _wspec(1, NODE_DIM), _wspec(1, NODE_DIM),
                  _wspec(NODE_DIM, NODE_DIM), _wspec(1, NODE_DIM),
                  _wspec(1, NODE_DIM), _wspec(1, NODE_DIM),
                  _wspec(NODE_DIM, INTER), _wspec(NODE_DIM, INTER),
                  _wspec(NODE_DIM, 16), _wspec(NODE_DIM, 16)],
        out_specs=[_rspec(BN, NODE_DIM), _rspec(BN, INTER), _rspec(BN, INTER),
                   _rspec(BN, 16), _rspec(BN, 16)],
        out_shape=(_f32(N_NODES, NODE_DIM), _f32(N_NODES, INTER),
                   _f32(N_NODES, INTER), _f32(N_NODES, 16),
                   _f32(N_NODES, 16)),
    )(x, hn, sega2, sega2, segb2, segb2, deg2, deg2, bm, wc, bc, o0, ob0,
      og, obb, o1, ob1, fg, fb, wtl, wtr, wcbl, wcbr)


P8 = 8
R8 = N_EDGES // P8
BEP = 2000


def _lnp(x, M, g, b):
    m = _dot(x, M)
    xc = x - m
    v = _dot(xc * xc, M)
    return xc * lax.rsqrt(v + 1e-5) * g + b


def _tc_edge_inter_kernel(hb, gl, gr, m32, bl, l0, lb0, lg, lbb, l1, lb1,
                          br, r0, rb0, rg, rbb, r1, rb1, mbl, mbr):
    hbb = hb[...]
    M = m32[...]
    il = _dot(hbb, bl[...]) + gl[...]
    h = _dot(il, l0[...]) + lb0[...]
    h = jnp.maximum(_lnp(h, M, lg[...], lbb[...]), 0.0)
    mbl[...] = _dot(h, l1[...]) + lb1[...]
    ir = _dot(hbb, br[...]) + gr[...]
    h = _dot(ir, r0[...]) + rb0[...]
    h = jnp.maximum(_lnp(h, M, rg[...], rbb[...]), 0.0)
    mbr[...] = _dot(h, r1[...]) + rb1[...]


def _tc_edge_inter(hb, gl, gr, *ws):
    return pl.pallas_call(
        _tc_edge_inter_kernel,
        grid=(R8 // BEP,),
        in_specs=[_rspec(BEP, 128), _rspec(BEP, 256), _rspec(BEP, 256),
                  _wspec(256, 256),
                  _wspec(128, 256), _wspec(256, 256),
                  _wspec(1, 256), _wspec(1, 256), _wspec(1, 256),
                  _wspec(256, 128), _wspec(1, 128),
                  _wspec(128, 256), _wspec(256, 256),
                  _wspec(1, 256), _wspec(1, 256), _wspec(1, 256),
                  _wspec(256, 128), _wspec(1, 128)],
        out_specs=[_rspec(BEP, 128), _rspec(BEP, 128)],
        out_shape=(_f32(R8, 128), _f32(R8, 128)),
    )(hb, gl, gr, *ws)


def _tc_edge_finish_kernel(gL, gR, hb, m16, ws, ball, o0, ob0, og, obb,
                           o1, ob1, fg, fb, out):
    hbb = hb[...]
    M = m16[...]
    upd = gL[...] + gR[...] + _dot(hbb, ws[...]) + ball[...]
    h = _dot(upd, o0[...]) + ob0[...]
    h = jnp.maximum(_lnp(h, M, og[...], obb[...]), 0.0)
    t = _dot(h, o1[...]) + ob1[...]
    out[...] = _lnp(t + hbb, M, fg[...], fb[...])


def _tc_edge_finish(gL, gR, hb, *ws):
    return pl.pallas_call(
        _tc_edge_finish_kernel,
        grid=(R8 // BEP,),
        in_specs=[_rspec(BEP, 128), _rspec(BEP, 128), _rspec(BEP, 128),
                  _wspec(128, 128),
                  _wspec(128, 128), _wspec(1, 128),
                  _wspec(128, 128), _wspec(1, 128),
                  _wspec(1, 128), _wspec(1, 128),
                  _wspec(128, 128), _wspec(1, 128),
                  _wspec(1, 128), _wspec(1, 128)],
        out_specs=_rspec(BEP, 128),
        out_shape=_f32(R8, 128),
    )(gL, gR, hb, *ws)


def _r1(v):
    return v.reshape(1, -1)


def _mlp_ws(p):
    return (p["l0"]["w"], _r1(p["l0"]["b"]), _r1(p["ln"]["g"]),
            _r1(p["ln"]["b"]), p["l1"]["w"], _r1(p["l1"]["b"]))


def _mlp_ws_fold(p, wm):
    return (p["l0"]["w"], _r1(p["l0"]["b"]), _r1(p["ln"]["g"]),
            _r1(p["ln"]["b"]), p["l1"]["w"] @ wm, _r1(p["l1"]["b"] @ wm))


def _bd(w):
    return jnp.kron(jnp.eye(P8, dtype=w.dtype), w)


def _t8(v):
    return jnp.tile(v.reshape(1, -1), (1, P8))


def _mlp_ws_p(p):
    return (_bd(p["l0"]["w"]), _t8(p["l0"]["b"]), _t8(p["ln"]["g"]),
            _t8(p["ln"]["b"]), _bd(p["l1"]["w"]), _t8(p["l1"]["b"]))


def _mlp_ws_fold_p(p, wm):
    return (_bd(p["l0"]["w"]), _t8(p["l0"]["b"]), _t8(p["ln"]["g"]),
            _t8(p["ln"]["b"]), _bd(p["l1"]["w"] @ wm), _t8(p["l1"]["b"] @ wm))


def kernel(h_node, h_edge, edge_index, node_extra, edge_extra, params):
    row3 = edge_index[0].reshape(NW, NB, BATCH)
    col3 = edge_index[1].reshape(NW, NB, BATCH)
    row16 = edge_index[0].reshape(NS, NBS, BATCH)
    col16 = edge_index[1].reshape(NS, NBS, BATCH)

    deg2 = _sc_deg(row3).reshape(2, N_PAD, 16)

    x, hb = h_node, h_edge
    for blk in params["blocks"]:
        npar, epar = blk["node"], blk["edge"]

        wm = npar["msg_net"]["w"]
        hn = _tc_node_mlp(x, *_mlp_ws_fold(npar["node_net"], wm))
        sega2 = _sc_agg_gather(hn, row3, col3).reshape(2, N_PAD, NODE_DIM)
        he = _tc_edge_mlp(hb, *_mlp_ws_fold(npar["edge_net"], wm))
        segb2 = _sc_agg_stream(he.reshape(NW * NB, BATCH, NODE_DIM),
                               row3, row3).reshape(2, N_PAD, NODE_DIM)
        x, tl, tr, cbl, cbr = _tc_node_finish(
            x, hn, sega2, segb2, deg2,
            _r1(npar["msg_net"]["b"]),
            npar["centroid_lin"]["w"], _r1(npar["centroid_lin"]["b"]),
            *_mlp_ws(npar["out_layer"]),
            _r1(npar["layer_norm"]["g"]), _r1(npar["layer_norm"]["b"]),
            epar["bond_ffn_left"]["node_linear"]["w"],
            epar["bond_ffn_right"]["node_linear"]["w"],
            epar["node_ffn_left"]["w"], epar["node_ffn_right"]["w"])

        gl3, gr3 = _sc_gather32(tl, tr, row3, col3)
        m32 = _bd(jnp.full((INTER, INTER), 1.0 / INTER, jnp.float32))
        mbl8, mbr8 = _tc_edge_inter(
            hb.reshape(R8, P8 * EDGE_DIM),
            gl3.reshape(R8, P8 * INTER), gr3.reshape(R8, P8 * INTER),
            m32,
            _bd(epar["bond_ffn_left"]["bond_linear"]["w"]),
            *_mlp_ws_fold_p(epar["bond_ffn_left"]["inter_module"],
                            epar["msg_left"]["w"]),
            _bd(epar["bond_ffn_right"]["bond_linear"]["w"]),
            *_mlp_ws_fold_p(epar["bond_ffn_right"]["inter_module"],
                            epar["msg_right"]["w"]))

        gL3, gR3 = _sc_scatter_gather(
            mbl8.reshape(NS * NBS, BATCH, EDGE_DIM),
            mbr8.reshape(NS * NBS, BATCH, EDGE_DIM),
            cbl, cbr, col16, row16, row16, col16)
        m16 = _bd(jnp.full((EDGE_DIM, EDGE_DIM), 1.0 / EDGE_DIM, jnp.float32))
        ball = _t8(epar["msg_left"]["b"] + epar["node_ffn_left"]["b"]
                   + epar["msg_right"]["b"] + epar["node_ffn_right"]["b"]
                   + epar["self_ffn"]["b"])
        hb = _tc_edge_finish(
            gL3.reshape(R8, P8 * EDGE_DIM), gR3.reshape(R8, P8 * EDGE_DIM),
            hb.reshape(R8, P8 * EDGE_DIM), m16,
            _bd(epar["self_ffn"]["w"]), ball,
            *_mlp_ws_p(epar["out_layer"]),
            _t8(epar["layer_norm"]["g"]),
            _t8(epar["layer_norm"]["b"])).reshape(N_EDGES, EDGE_DIM)

    return x, hb
